# baseline (speedup 1.0000x reference)
import functools

import jax
import jax.numpy as jnp
from jax import lax
from jax.experimental import pallas as pl
from jax.experimental.pallas import tpu as pltpu
from jax.experimental.pallas import tpu_sc as plsc

_BOX = 6.0
_INTERP = False


def _dot1(a, b):
    return lax.dot_general(a, b, (((1,), (0,)), ((), ())),
                           preferred_element_type=jnp.float32)


def _split(x):
    xh = x.astype(jnp.bfloat16)
    return xh, (x - xh.astype(jnp.float32)).astype(jnp.bfloat16)


def _dot(a, b):
    return _dot1(a.astype(jnp.bfloat16), b.astype(jnp.bfloat16))


def _b(x):
    return x.astype(jnp.bfloat16).astype(jnp.float32)


def _dot_oh(oh, b):
    bh, bl = _split(b)
    return _dot1(oh, bh) + _dot1(oh, bl)


def _pad2(x, r, c):
    return jnp.pad(x, ((0, r - x.shape[0]), (0, c - x.shape[1])))


def _padi(x, n, fill):
    return jnp.pad(x, (0, n - x.shape[0]), constant_values=fill).astype(jnp.int32)


def _prep_edge_w(lyrs, ds_f, dr_f, Dts, Dtr, D_out):
    W1 = lyrs[0]["W"]
    H1 = W1.shape[0]
    out = [_pad2(W1[:, 0:ds_f].T, Dts, H1),
           _pad2(W1[:, ds_f:ds_f + dr_f].T, Dtr, H1),
           W1[:, ds_f + dr_f][None, :],
           W1[:, ds_f + dr_f + 1][None, :],
           lyrs[0]["b"][None, :],
           W1[:, -1][None, :]]
    for i, lyr in enumerate(lyrs[1:]):
        WT, bb = lyr["W"].T, lyr["b"][None, :]
        if i == len(lyrs) - 2:
            WT, bb = _pad2(WT, WT.shape[0], D_out), _pad2(bb, 1, D_out)
        out += [WT, bb]
    return out


def _edge_mlp_body(feats_s, feats_r, u, w, ds_pos, dr_pos):
    W1sT, W1rT, p0, p1, b1, w1u = w[:6]
    rel = feats_s[:, ds_pos:ds_pos + 2] - feats_r[:, dr_pos:dr_pos + 2]
    rel = jnp.where(rel > _BOX / 2, rel - _BOX, rel)
    rel = jnp.where(rel <= -_BOX / 2, rel + _BOX, rel)
    h = _dot(feats_s, W1sT) + _dot(feats_r, W1rT)
    relb = _b(rel)
    h = (h + relb[:, 0:1] * _b(p0) + relb[:, 1:2] * _b(p1) + b1
         + _b(u) * _b(w1u))
    h = jnp.maximum(h, 0.0)
    for j in range(6, len(w), 2):
        h = jnp.maximum(_dot(h, w[j]) + w[j + 1], 0.0)
    return h


def _edge_onehot(idx_s, idx_r, table_s, table_r, u, wts, ds_pos, dr_pos, B,
                 D_out, idx_o=None, NB_out=None):
    E_pad = idx_s.shape[0]
    NBLK = E_pad // B
    NBs, NBr = table_s.shape[0], table_r.shape[0]
    nw = len(wts)
    scatter = idx_o is not None

    def kern(*refs):
        is_ref, ir_ref = refs[0], refs[1]
        k = 2
        if scatter:
            io_ref = refs[2]
            k = 3
        ts_ref, tr_ref, u_ref = refs[k:k + 3]
        w_refs = refs[k + 3:k + 3 + nw]
        out_ref = refs[k + 3 + nw]
        ib_s = is_ref[0]
        ib_r = ir_ref[0]
        oh_s = (lax.broadcasted_iota(jnp.int32, (B, NBs), 1) == ib_s
                ).astype(jnp.bfloat16)
        oh_r = (lax.broadcasted_iota(jnp.int32, (B, NBr), 1) == ib_r
                ).astype(jnp.bfloat16)
        feats_s = _dot_oh(oh_s, ts_ref[...])
        feats_r = _dot_oh(oh_r, tr_ref[...])
        w = [r[...] for r in w_refs]
        h = _edge_mlp_body(feats_s, feats_r, u_ref[0, 0], w, ds_pos, dr_pos)
        if scatter:
            ob = io_ref[0]
            oh_o = (lax.broadcasted_iota(jnp.int32, (NB_out, B), 0) == ob
                    ).astype(jnp.bfloat16)
            contrib = _dot_oh(oh_o, h)

            @pl.when(pl.program_id(0) == 0)
            def _():
                out_ref[...] = contrib

            @pl.when(pl.program_id(0) != 0)
            def _():
                out_ref[...] = out_ref[...] + contrib
        else:
            out_ref[...] = h[:, 0:D_out // 2]
            refs[k + 4 + nw][...] = h[:, D_out // 2:]

    in_specs = [pl.BlockSpec((1, B, 1), lambda i: (i, 0, 0)),
                pl.BlockSpec((1, B, 1), lambda i: (i, 0, 0))]
    args = [idx_s.reshape(NBLK, B, 1), idx_r.reshape(NBLK, B, 1)]
    if scatter:
        in_specs.append(pl.BlockSpec((1, 1, B), lambda i: (i, 0, 0)))
        args.append(idx_o.reshape(NBLK, 1, B))
    for a in (table_s, table_r, u, *wts):
        in_specs.append(pl.BlockSpec(a.shape, lambda i: (0, 0)))
        args.append(a)
    if scatter:
        out_shape = jax.ShapeDtypeStruct((NB_out, D_out), jnp.float32)
        out_spec = pl.BlockSpec((NB_out, D_out), lambda i: (0, 0))
    else:
        half = jax.ShapeDtypeStruct((E_pad, D_out // 2), jnp.float32)
        out_shape = [half, half]
        out_spec = [pl.BlockSpec((B, D_out // 2), lambda i: (i, 0))] * 2
    return pl.pallas_call(kern, grid=(NBLK,), in_specs=in_specs,
                          out_specs=out_spec, out_shape=out_shape,
                          interpret=_INTERP)(*args)


def _edge_rows(rows_s, rows_r, u, wts, ds_pos, dr_pos, B, D_out):
    E_pad = rows_s.shape[0]
    NBLK = E_pad // B
    nw = len(wts)

    def kern(*refs):
        rs_ref, rr_ref, u_ref = refs[0], refs[1], refs[2]
        w = [r[...] for r in refs[3:3 + nw]]
        h = _edge_mlp_body(rs_ref[...], rr_ref[...], u_ref[0, 0],
                           w, ds_pos, dr_pos)
        refs[3 + nw][...] = h[:, 0:D_out // 2]
        refs[4 + nw][...] = h[:, D_out // 2:]

    in_specs = [pl.BlockSpec((B, rows_s.shape[1]), lambda i: (i, 0)),
                pl.BlockSpec((B, rows_r.shape[1]), lambda i: (i, 0))]
    args = [rows_s, rows_r]
    for a in (u, *wts):
        in_specs.append(pl.BlockSpec(a.shape, lambda i: (0, 0)))
        args.append(a)
    half = jax.ShapeDtypeStruct((E_pad, D_out // 2), jnp.float32)
    return pl.pallas_call(
        kern, grid=(NBLK,), in_specs=in_specs,
        out_specs=[pl.BlockSpec((B, D_out // 2), lambda i: (i, 0))] * 2,
        out_shape=[half, half],
        interpret=_INTERP)(*args)


def _node_stage(table, parts, u, lyrs, dv, da, B, final=None):
    N, Dtab = table.shape
    Dagg = parts[0].shape[1]
    W1 = lyrs[0]["W"]
    H1 = W1.shape[0]
    wts = [_pad2(W1[:, 0:dv].T, Dtab, H1),
           _pad2(W1[:, dv:dv + da].T, Dagg, H1),
           lyrs[0]["b"][None, :],
           W1[:, -1][None, :]]
    for lyr in lyrs[1:]:
        wts += [lyr["W"].T, lyr["b"][None, :]]
    D_out = lyrs[-1]["W"].shape[0]
    if final is not None:
        wts += [_pad2(final["W"].T, final["W"].shape[1], 8),
                _pad2(final["b"][None, :], 1, 8)]
        D_out = 8
    nw = len(wts)
    nparts = len(parts)
    NBLK = N // B

    def kern(*refs):
        t_ref = refs[0]
        agg = refs[1][...]
        for j in range(2, 1 + nparts):
            agg = agg + refs[j][...]
        u_ref = refs[1 + nparts]
        w = [r[...] for r in refs[2 + nparts:2 + nparts + nw]]
        out_ref = refs[2 + nparts + nw]
        h = (_dot(t_ref[...], w[0]) + _dot(agg, w[1]) + w[2]
             + _b(u_ref[0, 0]) * _b(w[3]))
        h = jnp.maximum(h, 0.0)
        nl = len(lyrs) - 1
        k = 4
        for _ in range(nl):
            h = jnp.maximum(_dot(h, w[k]) + w[k + 1], 0.0)
            k += 2
        if final is not None:
            h = _dot(h, w[k]) + w[k + 1]
        out_ref[...] = h

    in_specs = [pl.BlockSpec((B, Dtab), lambda i: (i, 0))]
    args = [table]
    for p in parts:
        in_specs.append(pl.BlockSpec((B, Dagg), lambda i: (i, 0)))
        args.append(p)
    for a in (u, *wts):
        in_specs.append(pl.BlockSpec(a.shape, lambda i: (0, 0)))
        args.append(a)
    return pl.pallas_call(
        kern, grid=(NBLK,), in_specs=in_specs,
        out_specs=pl.BlockSpec((B, D_out), lambda i: (i, 0)),
        out_shape=jax.ShapeDtypeStruct((N, D_out), jnp.float32),
        interpret=_INTERP)(*args)


@functools.cache
def _sc_mesh():
    return plsc.VectorSubcoreMesh(core_axis_name="c", subcore_axis_name="s")


def _sc_gather_pair(table, idx_s, idx_r, CH):
    E_pad = idx_s.shape[0]
    N, D = table.shape
    per_w = E_pad // 32
    nch = per_w // CH

    @functools.partial(
        pl.kernel,
        out_type=[jax.ShapeDtypeStruct((E_pad, D), jnp.float32),
                  jax.ShapeDtypeStruct((E_pad, D), jnp.float32)],
        mesh=_sc_mesh(),
        compiler_params=pltpu.CompilerParams(use_tc_tiling_on_sc=False),
        scratch_types=[pltpu.VMEM((2, CH), jnp.int32),
                       pltpu.VMEM((2, CH), jnp.int32),
                       pltpu.VMEM((2, CH, D), jnp.float32),
                       pltpu.VMEM((2, CH, D), jnp.float32)]
        + [pltpu.SemaphoreType.DMA] * 6)
    def k(tab, isrc, irsc, outs, outr, iv_s, iv_r, rv_s, rv_r,
          si, sg, sw, si1, sg1, sw1):
        wid = lax.axis_index("s") * 2 + lax.axis_index("c")
        base = wid * per_w
        sem_i, sem_g, sem_w = (si, si1), (sg, sg1), (sw, sw1)

        def start_idx(j, b):
            off = base + j * CH
            return (pltpu.async_copy(isrc.at[pl.ds(off, CH)], iv_s.at[b],
                                     sem_i[b]),
                    pltpu.async_copy(irsc.at[pl.ds(off, CH)], iv_r.at[b],
                                     sem_i[b]))

        idx_d = [None, None]
        wb_d = [None, None]
        idx_d[0] = start_idx(0, 0)
        for j in range(nch):
            b = j & 1
            nb = 1 - b
            if j + 1 < nch:
                idx_d[nb] = start_idx(j + 1, nb)
            for c in idx_d[b]:
                c.wait()
            if wb_d[b] is not None:
                for c in wb_d[b]:
                    c.wait()
            cs = pltpu.async_copy(tab.at[iv_s.at[b]], rv_s.at[b], sem_g[b])
            cr = pltpu.async_copy(tab.at[iv_r.at[b]], rv_r.at[b], sem_g[b])
            cs.wait()
            cr.wait()
            off = base + j * CH
            wb_d[b] = (pltpu.async_copy(rv_s.at[b], outs.at[pl.ds(off, CH)],
                                        sem_w[b]),
                       pltpu.async_copy(rv_r.at[b], outr.at[pl.ds(off, CH)],
                                        sem_w[b]))
        for b in (0, 1):
            if wb_d[b] is not None:
                for c in wb_d[b]:
                    c.wait()

    return k(table, idx_s, idx_r)


def _sc_scatter_add(rows_list, idx_list, N, D, CH):
    stripe = N // 16
    per_w = [r.shape[0] // 32 for r in rows_list]
    nch = [p // CH for p in per_w]
    zeros = jnp.zeros((stripe, D), jnp.float32)

    seq = [(a, i) for a in range(len(rows_list)) for i in range(nch[a])]

    @functools.partial(
        pl.kernel,
        out_type=jax.ShapeDtypeStruct((2 * N, D), jnp.float32),
        mesh=_sc_mesh(),
        compiler_params=pltpu.CompilerParams(use_tc_tiling_on_sc=False),
        scratch_types=[pltpu.VMEM((2, CH), jnp.int32),
                       pltpu.VMEM((2, CH, D), jnp.float32),
                       pltpu.VMEM_SHARED((N, D), jnp.float32)]
        + [pltpu.SemaphoreType.DMA] * 4)
    def k(*refs):
        na = len(rows_list)
        rows = refs[0:na]
        idxs = refs[na:2 * na]
        zref = refs[2 * na]
        out = refs[2 * na + 1]
        iv, rv, acc, sl0, sl1, ss0, ss1 = refs[2 * na + 2:2 * na + 9]
        sem_l, sem_s = (sl0, sl1), (ss0, ss1)
        c = lax.axis_index("c")
        s = lax.axis_index("s")
        pltpu.sync_copy(zref, acc.at[pl.ds(s * stripe, stripe)])
        plsc.subcore_barrier()
        wid = s * 2 + c

        def start_load(j, b):
            a, i = seq[j]
            off = wid * per_w[a] + i * CH
            return (pltpu.async_copy(idxs[a].at[pl.ds(off, CH)], iv.at[b],
                                     sem_l[b]),
                    pltpu.async_copy(rows[a].at[pl.ds(off, CH)], rv.at[b],
                                     sem_l[b]))

        ld_d = [None, None]
        sc_d = [None, None]
        ld_d[0] = start_load(0, 0)
        for j in range(len(seq)):
            b = j & 1
            nb = 1 - b
            if j + 1 < len(seq):
                if sc_d[nb] is not None:
                    sc_d[nb].wait()
                    sc_d[nb] = None
                ld_d[nb] = start_load(j + 1, nb)
            for cpy in ld_d[b]:
                cpy.wait()
            if sc_d[b] is not None:
                sc_d[b].wait()
            sc_d[b] = pltpu.async_copy(rv.at[b], acc.at[iv.at[b]], sem_s[b],
                                       add=True)
        for b in (0, 1):
            if sc_d[b] is not None:
                sc_d[b].wait()
        plsc.subcore_barrier()
        pltpu.sync_copy(acc.at[pl.ds(s * stripe, stripe)],
                        out.at[pl.ds(c * N + s * stripe, stripe)])

    out = k(*rows_list, *idx_list, zeros)
    return out[0:N] + out[N:]


_NP_COLS = (0, 3, 4)


def kernel(V, R_s, R_r, assignments_0, assignments_1, V_supers_0, V_supers_1,
           super_graphs_0, super_graphs_1, dt, params):
    V = V[0]
    R_s = R_s[0].astype(jnp.int32)
    R_r = R_r[0].astype(jnp.int32)
    a0 = assignments_0[0].astype(jnp.int32)
    a1 = assignments_1[0].astype(jnp.int32)
    Vs0 = V_supers_0[0]
    Vs1 = V_supers_1[0]
    sg0 = super_graphs_0[0].astype(jnp.int32)
    sg1 = super_graphs_1[0].astype(jnp.int32)
    u = dt.reshape(1, 1).astype(jnp.float32)
    npc = jnp.array(_NP_COLS)

    Vnp, Vpos = V[:, npc], V[:, 1:3]
    Vs0np, Vs0pos = Vs0[:, npc], Vs0[:, 1:3]
    Vs1np, Vs1pos = Vs1[:, npc], Vs1[:, 1:3]
    Vtab = _pad2(jnp.concatenate([Vnp, Vpos], 1), 10240, 8)
    Vs0tab = _pad2(jnp.concatenate([Vs0np, Vs0pos], 1), 1024, 8)
    Vs1tab = _pad2(jnp.concatenate([Vs1np, Vs1pos], 1), 128, 8)

    w1 = _prep_edge_w(params["edge_to_super"], 3, 3, 8, 8, 112)
    agg1 = _edge_onehot(_padi(a0[:, 1], 10240, 0), _padi(a0[:, 0], 10240, 0),
                        Vtab[0:1024], Vs0tab, u, w1, 3, 3, B=512, D_out=112,
                        idx_o=_padi(a0[:, 0], 10240, 1023), NB_out=1024)
    VLtab = _pad2(jnp.concatenate([Vs0np, agg1[0:1000, 0:100], Vs0pos], 1),
                  1024, 112)

    w2 = _prep_edge_w(params["edge_to_upper"], 103, 3, 112, 8, 112)
    agg2 = _edge_onehot(_padi(a1[:, 1], 1024, 0), _padi(a1[:, 0], 1024, 0),
                        VLtab[0:128], Vs1tab, u, w2, 103, 3, B=512, D_out=112,
                        idx_o=_padi(a1[:, 0], 1024, 127), NB_out=128)
    Vtoptab = _pad2(jnp.concatenate([Vs1np, agg2[0:100, 0:100], Vs1pos], 1),
                    128, 112)

    w3 = _prep_edge_w(params["super_edge"], 103, 103, 112, 112, 160)
    agg3 = _edge_onehot(_padi(sg1[:, 0], 2048, 0), _padi(sg1[:, 1], 2048, 0),
                        Vtoptab, Vtoptab, u, w3, 103, 103, B=512, D_out=160,
                        idx_o=_padi(sg1[:, 1], 2048, 127), NB_out=128)
    vnew3 = _node_stage(Vtoptab, [agg3], u, params["super_node"], 103, 150,
                        B=128)
    Vuptab = _pad2(jnp.concatenate([Vs1np, vnew3[0:100, 0:100], Vs1pos], 1),
                   128, 112)

    w4a = _prep_edge_w(params["edge_from_upper"], 103, 103, 112, 112, 160)
    p4a = _edge_onehot(_padi(a1[:, 0], 1024, 0), _padi(a1[:, 1], 1024, 0),
                       Vuptab, VLtab[0:128], u, w4a, 103, 103, B=512,
                       D_out=160, idx_o=_padi(a1[:, 1], 1024, 1023),
                       NB_out=1024)
    p4b = _edge_onehot(_padi(sg0[:, 0], 16384, 0), _padi(sg0[:, 1], 16384, 0),
                       VLtab, VLtab, u, w3, 103, 103, B=512, D_out=160,
                       idx_o=_padi(sg0[:, 1], 16384, 1023), NB_out=1024)
    vnew4 = _node_stage(VLtab, [p4a, p4b], u, params["super_node"], 103, 150,
                        B=512)
    Vup2tab = _pad2(jnp.concatenate([Vs0np, vnew4[0:1000, 0:100], Vs0pos], 1),
                    1024, 112)

    gs, gr = _sc_gather_pair(Vtab, _padi(R_s, 163840, 0),
                             _padi(R_r, 163840, 0), CH=1024)
    w5e = _prep_edge_w(params["edge"], 3, 3, 8, 8, 160)
    en_lo, en_hi = _edge_rows(gs, gr, u, w5e, 3, 3, B=512, D_out=160)
    w5a = _prep_edge_w(params["edge_from_super"], 103, 3, 112, 8, 160)
    ui_lo, ui_hi = _edge_onehot(_padi(a0[:, 0], 10240, 0),
                                _padi(a0[:, 1], 10240, 0),
                                Vup2tab, Vtab[0:1024], u, w5a, 103, 3, B=512,
                                D_out=160)
    idx1 = _padi(R_r, 163840, 10200)
    idx2 = _padi(a0[:, 1], 10240, 10200)
    agg_lo = _sc_scatter_add([en_lo, ui_lo], [idx1, idx2], N=10240, D=80,
                             CH=320)
    agg_hi = _sc_scatter_add([en_hi, ui_hi], [idx1, idx2], N=10240, D=80,
                             CH=320)
    agg5 = jnp.concatenate([agg_lo, agg_hi], 1)
    out5 = _node_stage(Vtab, [agg5], u,
                       params["node"], 3, 150, B=512, final=params["linear"])
    return out5[0:10000, 0:4][None]


# speedup vs baseline: 1.0028x; 1.0028x over previous
import functools

import jax
import jax.numpy as jnp
from jax import lax
from jax.experimental import pallas as pl
from jax.experimental.pallas import tpu as pltpu
from jax.experimental.pallas import tpu_sc as plsc

_BOX = 6.0


def _dot1(a, b):
    return lax.dot_general(a, b, (((1,), (0,)), ((), ())),
                           preferred_element_type=jnp.float32)


def _split(x):
    xh = x.astype(jnp.bfloat16)
    return xh, (x - xh.astype(jnp.float32)).astype(jnp.bfloat16)


def _dot(a, b):
    return _dot1(a.astype(jnp.bfloat16), b.astype(jnp.bfloat16))


def _b(x):
    return x.astype(jnp.bfloat16).astype(jnp.float32)


def _dot_oh(oh, b):
    bh, bl = _split(b)
    return _dot1(oh, bh) + _dot1(oh, bl)


def _pad2(x, r, c):
    return jnp.pad(x, ((0, r - x.shape[0]), (0, c - x.shape[1])))


def _padi(x, n, fill):
    return jnp.pad(x, (0, n - x.shape[0]), constant_values=fill).astype(jnp.int32)


def _prep_edge_w(lyrs, ds_f, dr_f, Dts, Dtr, D_out):
    W1 = lyrs[0]["W"]
    H1 = W1.shape[0]
    out = [_pad2(W1[:, 0:ds_f].T, Dts, H1),
           _pad2(W1[:, ds_f:ds_f + dr_f].T, Dtr, H1),
           W1[:, ds_f + dr_f][None, :],
           W1[:, ds_f + dr_f + 1][None, :],
           lyrs[0]["b"][None, :],
           W1[:, -1][None, :]]
    for i, lyr in enumerate(lyrs[1:]):
        WT, bb = lyr["W"].T, lyr["b"][None, :]
        if i == len(lyrs) - 2:
            WT, bb = _pad2(WT, WT.shape[0], D_out), _pad2(bb, 1, D_out)
        out += [WT, bb]
    return out


def _edge_mlp_body(feats_s, feats_r, u, w, ds_pos, dr_pos):
    W1sT, W1rT, p0, p1, b1, w1u = w[:6]
    rel = feats_s[:, ds_pos:ds_pos + 2] - feats_r[:, dr_pos:dr_pos + 2]
    rel = jnp.where(rel > _BOX / 2, rel - _BOX, rel)
    rel = jnp.where(rel <= -_BOX / 2, rel + _BOX, rel)
    h = _dot(feats_s, W1sT) + _dot(feats_r, W1rT)
    relb = _b(rel)
    h = (h + relb[:, 0:1] * _b(p0) + relb[:, 1:2] * _b(p1) + b1
         + _b(u) * _b(w1u))
    h = jnp.maximum(h, 0.0)
    for j in range(6, len(w), 2):
        h = jnp.maximum(_dot(h, w[j]) + w[j + 1], 0.0)
    return h


def _edge_onehot(idx_s, idx_r, table_s, table_r, u, wts, ds_pos, dr_pos, B,
                 D_out, idx_o=None, NB_out=None):
    E_pad = idx_s.shape[0]
    NBLK = E_pad // B
    NBs, NBr = table_s.shape[0], table_r.shape[0]
    nw = len(wts)
    scatter = idx_o is not None

    def kern(*refs):
        is_ref, ir_ref = refs[0], refs[1]
        k = 2
        if scatter:
            io_ref = refs[2]
            k = 3
        ts_ref, tr_ref, u_ref = refs[k:k + 3]
        w_refs = refs[k + 3:k + 3 + nw]
        out_ref = refs[k + 3 + nw]
        ib_s = is_ref[0]
        ib_r = ir_ref[0]
        oh_s = (lax.broadcasted_iota(jnp.int32, (B, NBs), 1) == ib_s
                ).astype(jnp.bfloat16)
        oh_r = (lax.broadcasted_iota(jnp.int32, (B, NBr), 1) == ib_r
                ).astype(jnp.bfloat16)
        feats_s = _dot_oh(oh_s, ts_ref[...])
        feats_r = _dot_oh(oh_r, tr_ref[...])
        w = [r[...] for r in w_refs]
        h = _edge_mlp_body(feats_s, feats_r, u_ref[0, 0], w, ds_pos, dr_pos)
        if scatter:
            ob = io_ref[0]
            oh_o = (lax.broadcasted_iota(jnp.int32, (NB_out, B), 0) == ob
                    ).astype(jnp.bfloat16)
            contrib = _dot_oh(oh_o, h)

            @pl.when(pl.program_id(0) == 0)
            def _():
                out_ref[...] = contrib

            @pl.when(pl.program_id(0) != 0)
            def _():
                out_ref[...] = out_ref[...] + contrib
        else:
            out_ref[...] = h[:, 0:D_out // 2]
            refs[k + 4 + nw][...] = h[:, D_out // 2:]

    in_specs = [pl.BlockSpec((1, B, 1), lambda i: (i, 0, 0)),
                pl.BlockSpec((1, B, 1), lambda i: (i, 0, 0))]
    args = [idx_s.reshape(NBLK, B, 1), idx_r.reshape(NBLK, B, 1)]
    if scatter:
        in_specs.append(pl.BlockSpec((1, 1, B), lambda i: (i, 0, 0)))
        args.append(idx_o.reshape(NBLK, 1, B))
    for a in (table_s, table_r, u, *wts):
        in_specs.append(pl.BlockSpec(a.shape, lambda i: (0, 0)))
        args.append(a)
    if scatter:
        out_shape = jax.ShapeDtypeStruct((NB_out, D_out), jnp.float32)
        out_spec = pl.BlockSpec((NB_out, D_out), lambda i: (0, 0))
    else:
        half = jax.ShapeDtypeStruct((E_pad, D_out // 2), jnp.float32)
        out_shape = [half, half]
        out_spec = [pl.BlockSpec((B, D_out // 2), lambda i: (i, 0))] * 2
    return pl.pallas_call(kern, grid=(NBLK,), in_specs=in_specs,
                          out_specs=out_spec, out_shape=out_shape)(*args)


def _edge_rows(rows_s, rows_r, u, wts, ds_pos, dr_pos, B, D_out):
    E_pad = rows_s.shape[0]
    NBLK = E_pad // B
    nw = len(wts)

    def kern(*refs):
        rs_ref, rr_ref, u_ref = refs[0], refs[1], refs[2]
        w = [r[...] for r in refs[3:3 + nw]]
        h = _edge_mlp_body(rs_ref[...], rr_ref[...], u_ref[0, 0],
                           w, ds_pos, dr_pos)
        refs[3 + nw][...] = h[:, 0:D_out // 2]
        refs[4 + nw][...] = h[:, D_out // 2:]

    in_specs = [pl.BlockSpec((B, rows_s.shape[1]), lambda i: (i, 0)),
                pl.BlockSpec((B, rows_r.shape[1]), lambda i: (i, 0))]
    args = [rows_s, rows_r]
    for a in (u, *wts):
        in_specs.append(pl.BlockSpec(a.shape, lambda i: (0, 0)))
        args.append(a)
    half = jax.ShapeDtypeStruct((E_pad, D_out // 2), jnp.float32)
    return pl.pallas_call(
        kern, grid=(NBLK,), in_specs=in_specs,
        out_specs=[pl.BlockSpec((B, D_out // 2), lambda i: (i, 0))] * 2,
        out_shape=[half, half])(*args)


def _node_stage(table, parts, u, lyrs, dv, da, B, final=None):
    N, Dtab = table.shape
    Dagg = parts[0].shape[1]
    W1 = lyrs[0]["W"]
    H1 = W1.shape[0]
    wts = [_pad2(W1[:, 0:dv].T, Dtab, H1),
           _pad2(W1[:, dv:dv + da].T, Dagg, H1),
           lyrs[0]["b"][None, :],
           W1[:, -1][None, :]]
    for lyr in lyrs[1:]:
        wts += [lyr["W"].T, lyr["b"][None, :]]
    D_out = lyrs[-1]["W"].shape[0]
    if final is not None:
        wts += [_pad2(final["W"].T, final["W"].shape[1], 8),
                _pad2(final["b"][None, :], 1, 8)]
        D_out = 8
    nw = len(wts)
    nparts = len(parts)
    NBLK = N // B

    def kern(*refs):
        t_ref = refs[0]
        agg = refs[1][...]
        for j in range(2, 1 + nparts):
            agg = agg + refs[j][...]
        u_ref = refs[1 + nparts]
        w = [r[...] for r in refs[2 + nparts:2 + nparts + nw]]
        out_ref = refs[2 + nparts + nw]
        h = (_dot(t_ref[...], w[0]) + _dot(agg, w[1]) + w[2]
             + _b(u_ref[0, 0]) * _b(w[3]))
        h = jnp.maximum(h, 0.0)
        nl = len(lyrs) - 1
        k = 4
        for _ in range(nl):
            h = jnp.maximum(_dot(h, w[k]) + w[k + 1], 0.0)
            k += 2
        if final is not None:
            h = _dot(h, w[k]) + w[k + 1]
        out_ref[...] = h

    in_specs = [pl.BlockSpec((B, Dtab), lambda i: (i, 0))]
    args = [table]
    for p in parts:
        in_specs.append(pl.BlockSpec((B, Dagg), lambda i: (i, 0)))
        args.append(p)
    for a in (u, *wts):
        in_specs.append(pl.BlockSpec(a.shape, lambda i: (0, 0)))
        args.append(a)
    return pl.pallas_call(
        kern, grid=(NBLK,), in_specs=in_specs,
        out_specs=pl.BlockSpec((B, D_out), lambda i: (i, 0)),
        out_shape=jax.ShapeDtypeStruct((N, D_out), jnp.float32))(*args)


@functools.cache
def _sc_mesh():
    return plsc.VectorSubcoreMesh(core_axis_name="c", subcore_axis_name="s")


def _sc_gather_pair(table, idx_s, idx_r, CH):
    E_pad = idx_s.shape[0]
    N, D = table.shape
    per_w = E_pad // 32
    nch = per_w // CH

    @functools.partial(
        pl.kernel,
        out_type=[jax.ShapeDtypeStruct((E_pad, D), jnp.float32),
                  jax.ShapeDtypeStruct((E_pad, D), jnp.float32)],
        mesh=_sc_mesh(),
        compiler_params=pltpu.CompilerParams(use_tc_tiling_on_sc=False),
        scratch_types=[pltpu.VMEM((2, CH), jnp.int32),
                       pltpu.VMEM((2, CH), jnp.int32),
                       pltpu.VMEM((2, CH, D), jnp.float32),
                       pltpu.VMEM((2, CH, D), jnp.float32)]
        + [pltpu.SemaphoreType.DMA] * 6)
    def k(tab, isrc, irsc, outs, outr, iv_s, iv_r, rv_s, rv_r,
          si, sg, sw, si1, sg1, sw1):
        wid = lax.axis_index("s") * 2 + lax.axis_index("c")
        base = wid * per_w
        sem_i, sem_g, sem_w = (si, si1), (sg, sg1), (sw, sw1)

        def start_idx(j, b):
            off = base + j * CH
            return (pltpu.async_copy(isrc.at[pl.ds(off, CH)], iv_s.at[b],
                                     sem_i[b]),
                    pltpu.async_copy(irsc.at[pl.ds(off, CH)], iv_r.at[b],
                                     sem_i[b]))

        idx_d = [None, None]
        wb_d = [None, None]
        idx_d[0] = start_idx(0, 0)
        for j in range(nch):
            b = j & 1
            nb = 1 - b
            if j + 1 < nch:
                idx_d[nb] = start_idx(j + 1, nb)
            for c in idx_d[b]:
                c.wait()
            if wb_d[b] is not None:
                for c in wb_d[b]:
                    c.wait()
            cs = pltpu.async_copy(tab.at[iv_s.at[b]], rv_s.at[b], sem_g[b])
            cr = pltpu.async_copy(tab.at[iv_r.at[b]], rv_r.at[b], sem_g[b])
            cs.wait()
            cr.wait()
            off = base + j * CH
            wb_d[b] = (pltpu.async_copy(rv_s.at[b], outs.at[pl.ds(off, CH)],
                                        sem_w[b]),
                       pltpu.async_copy(rv_r.at[b], outr.at[pl.ds(off, CH)],
                                        sem_w[b]))
        for b in (0, 1):
            if wb_d[b] is not None:
                for c in wb_d[b]:
                    c.wait()

    return k(table, idx_s, idx_r)


def _sc_scatter_add(rows_list, idx_list, N, D, CH):
    stripe = N // 16
    per_w = [r.shape[0] // 32 for r in rows_list]
    nch = [p // CH for p in per_w]
    zeros = jnp.zeros((stripe, D), jnp.float32)

    seq = [(a, i) for a in range(len(rows_list)) for i in range(nch[a])]

    @functools.partial(
        pl.kernel,
        out_type=jax.ShapeDtypeStruct((2 * N, D), jnp.float32),
        mesh=_sc_mesh(),
        compiler_params=pltpu.CompilerParams(use_tc_tiling_on_sc=False),
        scratch_types=[pltpu.VMEM((2, CH), jnp.int32),
                       pltpu.VMEM((2, CH, D), jnp.float32),
                       pltpu.VMEM_SHARED((N, D), jnp.float32)]
        + [pltpu.SemaphoreType.DMA] * 4)
    def k(*refs):
        na = len(rows_list)
        rows = refs[0:na]
        idxs = refs[na:2 * na]
        zref = refs[2 * na]
        out = refs[2 * na + 1]
        iv, rv, acc, sl0, sl1, ss0, ss1 = refs[2 * na + 2:2 * na + 9]
        sem_l, sem_s = (sl0, sl1), (ss0, ss1)
        c = lax.axis_index("c")
        s = lax.axis_index("s")
        pltpu.sync_copy(zref, acc.at[pl.ds(s * stripe, stripe)])
        plsc.subcore_barrier()
        wid = s * 2 + c

        def start_load(j, b):
            a, i = seq[j]
            off = wid * per_w[a] + i * CH
            return (pltpu.async_copy(idxs[a].at[pl.ds(off, CH)], iv.at[b],
                                     sem_l[b]),
                    pltpu.async_copy(rows[a].at[pl.ds(off, CH)], rv.at[b],
                                     sem_l[b]))

        ld_d = [None, None]
        sc_d = [None, None]
        ld_d[0] = start_load(0, 0)
        for j in range(len(seq)):
            b = j & 1
            nb = 1 - b
            if j + 1 < len(seq):
                if sc_d[nb] is not None:
                    sc_d[nb].wait()
                    sc_d[nb] = None
                ld_d[nb] = start_load(j + 1, nb)
            for cpy in ld_d[b]:
                cpy.wait()
            if sc_d[b] is not None:
                sc_d[b].wait()
            sc_d[b] = pltpu.async_copy(rv.at[b], acc.at[iv.at[b]], sem_s[b],
                                       add=True)
        for b in (0, 1):
            if sc_d[b] is not None:
                sc_d[b].wait()
        plsc.subcore_barrier()
        pltpu.sync_copy(acc.at[pl.ds(s * stripe, stripe)],
                        out.at[pl.ds(c * N + s * stripe, stripe)])

    out = k(*rows_list, *idx_list, zeros)
    return out[0:N] + out[N:]


_NP_COLS = (0, 3, 4)


def kernel(V, R_s, R_r, assignments_0, assignments_1, V_supers_0, V_supers_1,
           super_graphs_0, super_graphs_1, dt, params):
    V = V[0]
    R_s = R_s[0].astype(jnp.int32)
    R_r = R_r[0].astype(jnp.int32)
    a0 = assignments_0[0].astype(jnp.int32)
    a1 = assignments_1[0].astype(jnp.int32)
    Vs0 = V_supers_0[0]
    Vs1 = V_supers_1[0]
    sg0 = super_graphs_0[0].astype(jnp.int32)
    sg1 = super_graphs_1[0].astype(jnp.int32)
    u = dt.reshape(1, 1).astype(jnp.float32)
    npc = jnp.array(_NP_COLS)

    Vnp, Vpos = V[:, npc], V[:, 1:3]
    Vs0np, Vs0pos = Vs0[:, npc], Vs0[:, 1:3]
    Vs1np, Vs1pos = Vs1[:, npc], Vs1[:, 1:3]
    Vtab = _pad2(jnp.concatenate([Vnp, Vpos], 1), 10240, 16)
    Vs0tab = _pad2(jnp.concatenate([Vs0np, Vs0pos], 1), 1024, 16)
    Vs1tab = _pad2(jnp.concatenate([Vs1np, Vs1pos], 1), 128, 16)

    w1 = _prep_edge_w(params["edge_to_super"], 3, 3, 16, 16, 112)
    agg1 = _edge_onehot(_padi(a0[:, 1], 10240, 0), _padi(a0[:, 0], 10240, 0),
                        Vtab[0:1024], Vs0tab, u, w1, 3, 3, B=512, D_out=112,
                        idx_o=_padi(a0[:, 0], 10240, 1023), NB_out=1024)
    VLtab = _pad2(jnp.concatenate([Vs0np, agg1[0:1000, 0:100], Vs0pos], 1),
                  1024, 112)

    w2 = _prep_edge_w(params["edge_to_upper"], 103, 3, 112, 16, 112)
    agg2 = _edge_onehot(_padi(a1[:, 1], 1024, 0), _padi(a1[:, 0], 1024, 0),
                        VLtab[0:128], Vs1tab, u, w2, 103, 3, B=512, D_out=112,
                        idx_o=_padi(a1[:, 0], 1024, 127), NB_out=128)
    Vtoptab = _pad2(jnp.concatenate([Vs1np, agg2[0:100, 0:100], Vs1pos], 1),
                    128, 112)

    w3 = _prep_edge_w(params["super_edge"], 103, 103, 112, 112, 160)
    agg3 = _edge_onehot(_padi(sg1[:, 0], 2048, 0), _padi(sg1[:, 1], 2048, 0),
                        Vtoptab, Vtoptab, u, w3, 103, 103, B=512, D_out=160,
                        idx_o=_padi(sg1[:, 1], 2048, 127), NB_out=128)
    vnew3 = _node_stage(Vtoptab, [agg3], u, params["super_node"], 103, 150,
                        B=128)
    Vuptab = _pad2(jnp.concatenate([Vs1np, vnew3[0:100, 0:100], Vs1pos], 1),
                   128, 112)

    w4a = _prep_edge_w(params["edge_from_upper"], 103, 103, 112, 112, 160)
    p4a = _edge_onehot(_padi(a1[:, 0], 1024, 0), _padi(a1[:, 1], 1024, 0),
                       Vuptab, VLtab[0:128], u, w4a, 103, 103, B=512,
                       D_out=160, idx_o=_padi(a1[:, 1], 1024, 1023),
                       NB_out=1024)
    p4b = _edge_onehot(_padi(sg0[:, 0], 16384, 0), _padi(sg0[:, 1], 16384, 0),
                       VLtab, VLtab, u, w3, 103, 103, B=512, D_out=160,
                       idx_o=_padi(sg0[:, 1], 16384, 1023), NB_out=1024)
    vnew4 = _node_stage(VLtab, [p4a, p4b], u, params["super_node"], 103, 150,
                        B=512)
    Vup2tab = _pad2(jnp.concatenate([Vs0np, vnew4[0:1000, 0:100], Vs0pos], 1),
                    1024, 112)

    gs, gr = _sc_gather_pair(Vtab, _padi(R_s, 163840, 0),
                             _padi(R_r, 163840, 0), CH=1024)
    w5e = _prep_edge_w(params["edge"], 3, 3, 16, 16, 160)
    en_lo, en_hi = _edge_rows(gs, gr, u, w5e, 3, 3, B=512, D_out=160)
    w5a = _prep_edge_w(params["edge_from_super"], 103, 3, 112, 16, 160)
    ui_lo, ui_hi = _edge_onehot(_padi(a0[:, 0], 10240, 0),
                                _padi(a0[:, 1], 10240, 0),
                                Vup2tab, Vtab[0:1024], u, w5a, 103, 3, B=512,
                                D_out=160)
    idx1 = _padi(R_r, 163840, 10200)
    idx2 = _padi(a0[:, 1], 10240, 10200)
    agg_lo = _sc_scatter_add([en_lo, ui_lo], [idx1, idx2], N=10240, D=80,
                             CH=320)
    agg_hi = _sc_scatter_add([en_hi, ui_hi], [idx1, idx2], N=10240, D=80,
                             CH=320)
    agg5 = jnp.concatenate([agg_lo, agg_hi], 1)
    out5 = _node_stage(Vtab, [agg5], u,
                       params["node"], 3, 150, B=512, final=params["linear"])
    return out5[0:10000, 0:4][None]


# speedup vs baseline: 1.0638x; 1.0608x over previous
import functools

import jax
import jax.numpy as jnp
from jax import lax
from jax.experimental import pallas as pl
from jax.experimental.pallas import tpu as pltpu
from jax.experimental.pallas import tpu_sc as plsc

_BOX = 6.0


def _dot1(a, b):
    return lax.dot_general(a, b, (((1,), (0,)), ((), ())),
                           preferred_element_type=jnp.float32)


def _split(x):
    xh = x.astype(jnp.bfloat16)
    return xh, (x - xh.astype(jnp.float32)).astype(jnp.bfloat16)


def _dot(a, b):
    return _dot1(a.astype(jnp.bfloat16), b.astype(jnp.bfloat16))


def _b(x):
    return x.astype(jnp.bfloat16).astype(jnp.float32)


def _dot_oh(oh, b):
    bh, bl = _split(b)
    return _dot1(oh, bh) + _dot1(oh, bl)


def _pad2(x, r, c):
    return jnp.pad(x, ((0, r - x.shape[0]), (0, c - x.shape[1])))


def _padi(x, n, fill):
    return jnp.pad(x, (0, n - x.shape[0]), constant_values=fill).astype(jnp.int32)


def _prep_edge_w(lyrs, ds_f, dr_f, Dts, Dtr, D_out):
    W1 = lyrs[0]["W"]
    H1 = W1.shape[0]
    out = [_pad2(W1[:, 0:ds_f].T, Dts, H1),
           _pad2(W1[:, ds_f:ds_f + dr_f].T, Dtr, H1),
           W1[:, ds_f + dr_f][None, :],
           W1[:, ds_f + dr_f + 1][None, :],
           lyrs[0]["b"][None, :],
           W1[:, -1][None, :]]
    for i, lyr in enumerate(lyrs[1:]):
        WT, bb = lyr["W"].T, lyr["b"][None, :]
        if i == len(lyrs) - 2:
            WT, bb = _pad2(WT, WT.shape[0], D_out), _pad2(bb, 1, D_out)
        out += [WT, bb]
    return out


def _edge_mlp_body(feats_s, feats_r, u, w, ds_pos, dr_pos):
    W1sT, W1rT, p0, p1, b1, w1u = w[:6]
    rel = feats_s[:, ds_pos:ds_pos + 2] - feats_r[:, dr_pos:dr_pos + 2]
    rel = jnp.where(rel > _BOX / 2, rel - _BOX, rel)
    rel = jnp.where(rel <= -_BOX / 2, rel + _BOX, rel)
    h = _dot(feats_s, W1sT) + _dot(feats_r, W1rT)
    relb = _b(rel)
    h = (h + relb[:, 0:1] * _b(p0) + relb[:, 1:2] * _b(p1) + b1
         + _b(u) * _b(w1u))
    h = jnp.maximum(h, 0.0)
    for j in range(6, len(w), 2):
        h = jnp.maximum(_dot(h, w[j]) + w[j + 1], 0.0)
    return h


def _edge_onehot(idx_s, idx_r, table_s, table_r, u, wts, ds_pos, dr_pos, B,
                 D_out, idx_o=None, NB_out=None):
    E_pad = idx_s.shape[0]
    NBLK = E_pad // B
    NBs, NBr = table_s.shape[0], table_r.shape[0]
    nw = len(wts)
    scatter = idx_o is not None

    def kern(*refs):
        is_ref, ir_ref = refs[0], refs[1]
        k = 2
        if scatter:
            io_ref = refs[2]
            k = 3
        ts_ref, tr_ref, u_ref = refs[k:k + 3]
        w_refs = refs[k + 3:k + 3 + nw]
        out_ref = refs[k + 3 + nw]
        ib_s = is_ref[0]
        ib_r = ir_ref[0]
        oh_s = (lax.broadcasted_iota(jnp.int32, (B, NBs), 1) == ib_s
                ).astype(jnp.bfloat16)
        oh_r = (lax.broadcasted_iota(jnp.int32, (B, NBr), 1) == ib_r
                ).astype(jnp.bfloat16)
        feats_s = _dot_oh(oh_s, ts_ref[...])
        feats_r = _dot_oh(oh_r, tr_ref[...])
        w = [r[...] for r in w_refs]
        h = _edge_mlp_body(feats_s, feats_r, u_ref[0, 0], w, ds_pos, dr_pos)
        if scatter:
            ob = io_ref[0]
            oh_o = (lax.broadcasted_iota(jnp.int32, (NB_out, B), 0) == ob
                    ).astype(jnp.bfloat16)
            contrib = _dot_oh(oh_o, h)

            @pl.when(pl.program_id(0) == 0)
            def _():
                out_ref[...] = contrib

            @pl.when(pl.program_id(0) != 0)
            def _():
                out_ref[...] = out_ref[...] + contrib
        else:
            out_ref[...] = h[:, 0:D_out // 2]
            refs[k + 4 + nw][...] = h[:, D_out // 2:]

    in_specs = [pl.BlockSpec((1, B, 1), lambda i: (i, 0, 0)),
                pl.BlockSpec((1, B, 1), lambda i: (i, 0, 0))]
    args = [idx_s.reshape(NBLK, B, 1), idx_r.reshape(NBLK, B, 1)]
    if scatter:
        in_specs.append(pl.BlockSpec((1, 1, B), lambda i: (i, 0, 0)))
        args.append(idx_o.reshape(NBLK, 1, B))
    for a in (table_s, table_r, u, *wts):
        in_specs.append(pl.BlockSpec(a.shape, lambda i: (0, 0)))
        args.append(a)
    if scatter:
        out_shape = jax.ShapeDtypeStruct((NB_out, D_out), jnp.float32)
        out_spec = pl.BlockSpec((NB_out, D_out), lambda i: (0, 0))
    else:
        half = jax.ShapeDtypeStruct((E_pad, D_out // 2), jnp.float32)
        out_shape = [half, half]
        out_spec = [pl.BlockSpec((B, D_out // 2), lambda i: (i, 0))] * 2
    return pl.pallas_call(kern, grid=(NBLK,), in_specs=in_specs,
                          out_specs=out_spec, out_shape=out_shape)(*args)


def _edge_rows(rows_s, rows_r, u, wts, ds_pos, dr_pos, B, D_out):
    E_pad = rows_s.shape[0]
    NBLK = E_pad // B
    nw = len(wts)

    def kern(*refs):
        rs_ref, rr_ref, u_ref = refs[0], refs[1], refs[2]
        w = [r[...] for r in refs[3:3 + nw]]
        h = _edge_mlp_body(rs_ref[...], rr_ref[...], u_ref[0, 0],
                           w, ds_pos, dr_pos)
        refs[3 + nw][...] = h[:, 0:D_out // 2]
        refs[4 + nw][...] = h[:, D_out // 2:]

    in_specs = [pl.BlockSpec((B, rows_s.shape[1]), lambda i: (i, 0)),
                pl.BlockSpec((B, rows_r.shape[1]), lambda i: (i, 0))]
    args = [rows_s, rows_r]
    for a in (u, *wts):
        in_specs.append(pl.BlockSpec(a.shape, lambda i: (0, 0)))
        args.append(a)
    half = jax.ShapeDtypeStruct((E_pad, D_out // 2), jnp.float32)
    return pl.pallas_call(
        kern, grid=(NBLK,), in_specs=in_specs,
        out_specs=[pl.BlockSpec((B, D_out // 2), lambda i: (i, 0))] * 2,
        out_shape=[half, half])(*args)


def _node_stage(table, parts, u, lyrs, dv, da, B, final=None):
    N, Dtab = table.shape
    Dagg = parts[0].shape[1]
    W1 = lyrs[0]["W"]
    H1 = W1.shape[0]
    wts = [_pad2(W1[:, 0:dv].T, Dtab, H1),
           _pad2(W1[:, dv:dv + da].T, Dagg, H1),
           lyrs[0]["b"][None, :],
           W1[:, -1][None, :]]
    for lyr in lyrs[1:]:
        wts += [lyr["W"].T, lyr["b"][None, :]]
    D_out = lyrs[-1]["W"].shape[0]
    if final is not None:
        wts += [_pad2(final["W"].T, final["W"].shape[1], 8),
                _pad2(final["b"][None, :], 1, 8)]
        D_out = 8
    nw = len(wts)
    nparts = len(parts)
    NBLK = N // B

    def kern(*refs):
        t_ref = refs[0]
        agg = refs[1][...]
        for j in range(2, 1 + nparts):
            agg = agg + refs[j][...]
        u_ref = refs[1 + nparts]
        w = [r[...] for r in refs[2 + nparts:2 + nparts + nw]]
        out_ref = refs[2 + nparts + nw]
        h = (_dot(t_ref[...], w[0]) + _dot(agg, w[1]) + w[2]
             + _b(u_ref[0, 0]) * _b(w[3]))
        h = jnp.maximum(h, 0.0)
        nl = len(lyrs) - 1
        k = 4
        for _ in range(nl):
            h = jnp.maximum(_dot(h, w[k]) + w[k + 1], 0.0)
            k += 2
        if final is not None:
            h = _dot(h, w[k]) + w[k + 1]
        out_ref[...] = h

    in_specs = [pl.BlockSpec((B, Dtab), lambda i: (i, 0))]
    args = [table]
    for p in parts:
        in_specs.append(pl.BlockSpec((B, Dagg), lambda i: (i, 0)))
        args.append(p)
    for a in (u, *wts):
        in_specs.append(pl.BlockSpec(a.shape, lambda i: (0, 0)))
        args.append(a)
    return pl.pallas_call(
        kern, grid=(NBLK,), in_specs=in_specs,
        out_specs=pl.BlockSpec((B, D_out), lambda i: (i, 0)),
        out_shape=jax.ShapeDtypeStruct((N, D_out), jnp.float32))(*args)


@functools.cache
def _sc_mesh():
    return plsc.VectorSubcoreMesh(core_axis_name="c", subcore_axis_name="s")


def _sc_gather_pair(table, idx_s, idx_r, CH):
    E_pad = idx_s.shape[0]
    N, D = table.shape
    per_w = E_pad // 32
    nch = per_w // CH

    @functools.partial(
        pl.kernel,
        out_type=[jax.ShapeDtypeStruct((E_pad, D), jnp.float32),
                  jax.ShapeDtypeStruct((E_pad, D), jnp.float32)],
        mesh=_sc_mesh(),
        compiler_params=pltpu.CompilerParams(use_tc_tiling_on_sc=False),
        scratch_types=[pltpu.VMEM((2, CH), jnp.int32),
                       pltpu.VMEM((2, CH), jnp.int32),
                       pltpu.VMEM((2, CH, D), jnp.float32),
                       pltpu.VMEM((2, CH, D), jnp.float32)]
        + [pltpu.SemaphoreType.DMA] * 6)
    def k(tab, isrc, irsc, outs, outr, iv_s, iv_r, rv_s, rv_r,
          si, sg, sw, si1, sg1, sw1):
        wid = lax.axis_index("s") * 2 + lax.axis_index("c")
        base = wid * per_w
        sem_i, sem_g, sem_w = (si, si1), (sg, sg1), (sw, sw1)

        def start_idx(j, b):
            off = base + j * CH
            return (pltpu.async_copy(isrc.at[pl.ds(off, CH)], iv_s.at[b],
                                     sem_i[b]),
                    pltpu.async_copy(irsc.at[pl.ds(off, CH)], iv_r.at[b],
                                     sem_i[b]))

        idx_d = [None, None]
        wb_d = [None, None]
        idx_d[0] = start_idx(0, 0)
        for j in range(nch):
            b = j & 1
            nb = 1 - b
            if j + 1 < nch:
                idx_d[nb] = start_idx(j + 1, nb)
            for c in idx_d[b]:
                c.wait()
            if wb_d[b] is not None:
                for c in wb_d[b]:
                    c.wait()
            cs = pltpu.async_copy(tab.at[iv_s.at[b]], rv_s.at[b], sem_g[b])
            cr = pltpu.async_copy(tab.at[iv_r.at[b]], rv_r.at[b], sem_g[b])
            cs.wait()
            cr.wait()
            off = base + j * CH
            wb_d[b] = (pltpu.async_copy(rv_s.at[b], outs.at[pl.ds(off, CH)],
                                        sem_w[b]),
                       pltpu.async_copy(rv_r.at[b], outr.at[pl.ds(off, CH)],
                                        sem_w[b]))
        for b in (0, 1):
            if wb_d[b] is not None:
                for c in wb_d[b]:
                    c.wait()

    return k(table, idx_s, idx_r)


def _sc_scatter_add(rows_list, idx_list, N, D, CH):
    stripe = N // 16
    per_w = [r.shape[0] // 32 for r in rows_list]
    nch = [p // CH for p in per_w]
    zeros = jnp.zeros((stripe, D), jnp.float32)

    seq = [(a, i) for a in range(len(rows_list)) for i in range(nch[a])]

    @functools.partial(
        pl.kernel,
        out_type=jax.ShapeDtypeStruct((2 * N, D), jnp.float32),
        mesh=_sc_mesh(),
        compiler_params=pltpu.CompilerParams(use_tc_tiling_on_sc=False),
        scratch_types=[pltpu.VMEM((2, CH), jnp.int32),
                       pltpu.VMEM((2, CH, D), jnp.float32),
                       pltpu.VMEM_SHARED((N, D), jnp.float32)]
        + [pltpu.SemaphoreType.DMA] * 4)
    def k(*refs):
        na = len(rows_list)
        rows = refs[0:na]
        idxs = refs[na:2 * na]
        zref = refs[2 * na]
        out = refs[2 * na + 1]
        iv, rv, acc, sl0, sl1, ss0, ss1 = refs[2 * na + 2:2 * na + 9]
        sem_l, sem_s = (sl0, sl1), (ss0, ss1)
        c = lax.axis_index("c")
        s = lax.axis_index("s")
        pltpu.sync_copy(zref, acc.at[pl.ds(s * stripe, stripe)])
        plsc.subcore_barrier()
        wid = s * 2 + c

        def start_load(j, b):
            a, i = seq[j]
            off = wid * per_w[a] + i * CH
            return (pltpu.async_copy(idxs[a].at[pl.ds(off, CH)], iv.at[b],
                                     sem_l[b]),
                    pltpu.async_copy(rows[a].at[pl.ds(off, CH)], rv.at[b],
                                     sem_l[b]))

        ld_d = [None, None]
        sc_d = [None, None]
        ld_d[0] = start_load(0, 0)
        for j in range(len(seq)):
            b = j & 1
            nb = 1 - b
            if j + 1 < len(seq):
                if sc_d[nb] is not None:
                    sc_d[nb].wait()
                    sc_d[nb] = None
                ld_d[nb] = start_load(j + 1, nb)
            for cpy in ld_d[b]:
                cpy.wait()
            if sc_d[b] is not None:
                sc_d[b].wait()
            sc_d[b] = pltpu.async_copy(rv.at[b], acc.at[iv.at[b]], sem_s[b],
                                       add=True)
        for b in (0, 1):
            if sc_d[b] is not None:
                sc_d[b].wait()
        plsc.subcore_barrier()
        pltpu.sync_copy(acc.at[pl.ds(s * stripe, stripe)],
                        out.at[pl.ds(c * N + s * stripe, stripe)])

    out = k(*rows_list, *idx_list, zeros)
    return out[0:N] + out[N:]


_NP_COLS = (0, 3, 4)


def kernel(V, R_s, R_r, assignments_0, assignments_1, V_supers_0, V_supers_1,
           super_graphs_0, super_graphs_1, dt, params):
    V = V[0]
    R_s = R_s[0].astype(jnp.int32)
    R_r = R_r[0].astype(jnp.int32)
    a0 = assignments_0[0].astype(jnp.int32)
    a1 = assignments_1[0].astype(jnp.int32)
    Vs0 = V_supers_0[0]
    Vs1 = V_supers_1[0]
    sg0 = super_graphs_0[0].astype(jnp.int32)
    sg1 = super_graphs_1[0].astype(jnp.int32)
    u = dt.reshape(1, 1).astype(jnp.float32)
    npc = jnp.array(_NP_COLS)

    Vnp, Vpos = V[:, npc], V[:, 1:3]
    Vs0np, Vs0pos = Vs0[:, npc], Vs0[:, 1:3]
    Vs1np, Vs1pos = Vs1[:, npc], Vs1[:, 1:3]
    Vtab = _pad2(jnp.concatenate([Vnp, Vpos], 1), 10240, 16)
    Vs0tab = _pad2(jnp.concatenate([Vs0np, Vs0pos], 1), 1024, 16)
    Vs1tab = _pad2(jnp.concatenate([Vs1np, Vs1pos], 1), 128, 16)

    w1 = _prep_edge_w(params["edge_to_super"], 3, 3, 16, 16, 112)
    agg1 = _edge_onehot(_padi(a0[:, 1], 10240, 0), _padi(a0[:, 0], 10240, 0),
                        Vtab[0:1024], Vs0tab, u, w1, 3, 3, B=512, D_out=112,
                        idx_o=_padi(a0[:, 0], 10240, 1023), NB_out=1024)
    VLtab = _pad2(jnp.concatenate([Vs0np, agg1[0:1000, 0:100], Vs0pos], 1),
                  1024, 112)

    w2 = _prep_edge_w(params["edge_to_upper"], 103, 3, 112, 16, 112)
    agg2 = _edge_onehot(_padi(a1[:, 1], 1024, 0), _padi(a1[:, 0], 1024, 0),
                        VLtab[0:128], Vs1tab, u, w2, 103, 3, B=512, D_out=112,
                        idx_o=_padi(a1[:, 0], 1024, 127), NB_out=128)
    Vtoptab = _pad2(jnp.concatenate([Vs1np, agg2[0:100, 0:100], Vs1pos], 1),
                    128, 112)

    w3 = _prep_edge_w(params["super_edge"], 103, 103, 112, 112, 160)
    agg3 = _edge_onehot(_padi(sg1[:, 0], 2048, 0), _padi(sg1[:, 1], 2048, 0),
                        Vtoptab, Vtoptab, u, w3, 103, 103, B=512, D_out=160,
                        idx_o=_padi(sg1[:, 1], 2048, 127), NB_out=128)
    vnew3 = _node_stage(Vtoptab, [agg3], u, params["super_node"], 103, 150,
                        B=128)
    Vuptab = _pad2(jnp.concatenate([Vs1np, vnew3[0:100, 0:100], Vs1pos], 1),
                   128, 112)

    w4a = _prep_edge_w(params["edge_from_upper"], 103, 103, 112, 112, 160)
    p4a = _edge_onehot(_padi(a1[:, 0], 1024, 0), _padi(a1[:, 1], 1024, 0),
                       Vuptab, VLtab[0:128], u, w4a, 103, 103, B=512,
                       D_out=160, idx_o=_padi(a1[:, 1], 1024, 1023),
                       NB_out=1024)
    p4b = _edge_onehot(_padi(sg0[:, 0], 16384, 0), _padi(sg0[:, 1], 16384, 0),
                       VLtab, VLtab, u, w3, 103, 103, B=1024, D_out=160,
                       idx_o=_padi(sg0[:, 1], 16384, 1023), NB_out=1024)
    vnew4 = _node_stage(VLtab, [p4a, p4b], u, params["super_node"], 103, 150,
                        B=512)
    Vup2tab = _pad2(jnp.concatenate([Vs0np, vnew4[0:1000, 0:100], Vs0pos], 1),
                    1024, 112)

    gs, gr = _sc_gather_pair(Vtab, _padi(R_s, 163840, 0),
                             _padi(R_r, 163840, 0), CH=1280)
    w5e = _prep_edge_w(params["edge"], 3, 3, 16, 16, 160)
    en_lo, en_hi = _edge_rows(gs, gr, u, w5e, 3, 3, B=1024, D_out=160)
    w5a = _prep_edge_w(params["edge_from_super"], 103, 3, 112, 16, 160)
    ui_lo, ui_hi = _edge_onehot(_padi(a0[:, 0], 10240, 0),
                                _padi(a0[:, 1], 10240, 0),
                                Vup2tab, Vtab[0:1024], u, w5a, 103, 3, B=1024,
                                D_out=160)
    idx1 = _padi(R_r, 163840, 10200)
    idx2 = _padi(a0[:, 1], 10240, 10200)
    agg_lo = _sc_scatter_add([en_lo, ui_lo], [idx1, idx2], N=10240, D=80,
                             CH=320)
    agg_hi = _sc_scatter_add([en_hi, ui_hi], [idx1, idx2], N=10240, D=80,
                             CH=320)
    agg5 = jnp.concatenate([agg_lo, agg_hi], 1)
    out5 = _node_stage(Vtab, [agg5], u,
                       params["node"], 3, 150, B=512, final=params["linear"])
    return out5[0:10000, 0:4][None]


# speedup vs baseline: 1.1009x; 1.0349x over previous
import functools

import jax
import jax.numpy as jnp
from jax import lax
from jax.experimental import pallas as pl
from jax.experimental.pallas import tpu as pltpu
from jax.experimental.pallas import tpu_sc as plsc

_BOX = 6.0


def _dot1(a, b):
    return lax.dot_general(a, b, (((1,), (0,)), ((), ())),
                           preferred_element_type=jnp.float32)


def _split(x):
    xh = x.astype(jnp.bfloat16)
    return xh, (x - xh.astype(jnp.float32)).astype(jnp.bfloat16)


def _dot(a, b):
    return _dot1(a.astype(jnp.bfloat16), b.astype(jnp.bfloat16))


def _b(x):
    return x.astype(jnp.bfloat16).astype(jnp.float32)


def _dot_oh(oh, b):
    bh, bl = _split(b)
    return _dot1(oh, bh) + _dot1(oh, bl)


def _pad2(x, r, c):
    return jnp.pad(x, ((0, r - x.shape[0]), (0, c - x.shape[1])))


def _padi(x, n, fill):
    return jnp.pad(x, (0, n - x.shape[0]), constant_values=fill).astype(jnp.int32)


def _prep_edge_w(lyrs, ds_f, dr_f, Dts, Dtr, D_out):
    W1 = lyrs[0]["W"]
    H1 = W1.shape[0]
    out = [_pad2(W1[:, 0:ds_f].T, Dts, H1),
           _pad2(W1[:, ds_f:ds_f + dr_f].T, Dtr, H1),
           W1[:, ds_f + dr_f][None, :],
           W1[:, ds_f + dr_f + 1][None, :],
           lyrs[0]["b"][None, :],
           W1[:, -1][None, :]]
    for i, lyr in enumerate(lyrs[1:]):
        WT, bb = lyr["W"].T, lyr["b"][None, :]
        if i == len(lyrs) - 2:
            WT, bb = _pad2(WT, WT.shape[0], D_out), _pad2(bb, 1, D_out)
        out += [WT, bb]
    return out


def _edge_mlp_body(feats_s, feats_r, u, w, ds_pos, dr_pos):
    W1sT, W1rT, p0, p1, b1, w1u = w[:6]
    rel = feats_s[:, ds_pos:ds_pos + 2] - feats_r[:, dr_pos:dr_pos + 2]
    rel = jnp.where(rel > _BOX / 2, rel - _BOX, rel)
    rel = jnp.where(rel <= -_BOX / 2, rel + _BOX, rel)
    h = _dot(feats_s, W1sT) + _dot(feats_r, W1rT)
    relb = _b(rel)
    h = (h + relb[:, 0:1] * _b(p0) + relb[:, 1:2] * _b(p1) + b1
         + _b(u) * _b(w1u))
    h = jnp.maximum(h, 0.0)
    for j in range(6, len(w), 2):
        h = jnp.maximum(_dot(h, w[j]) + w[j + 1], 0.0)
    return h


def _edge_onehot(idx_s, idx_r, table_s, table_r, u, wts, ds_pos, dr_pos, B,
                 D_out, idx_o=None, NB_out=None):
    E_pad = idx_s.shape[0]
    NBLK = E_pad // B
    NBs, NBr = table_s.shape[0], table_r.shape[0]
    nw = len(wts)
    scatter = idx_o is not None

    def kern(*refs):
        is_ref, ir_ref = refs[0], refs[1]
        k = 2
        if scatter:
            io_ref = refs[2]
            k = 3
        ts_ref, tr_ref, u_ref = refs[k:k + 3]
        w_refs = refs[k + 3:k + 3 + nw]
        out_ref = refs[k + 3 + nw]
        ib_s = is_ref[0]
        ib_r = ir_ref[0]
        oh_s = (lax.broadcasted_iota(jnp.int32, (B, NBs), 1) == ib_s
                ).astype(jnp.bfloat16)
        oh_r = (lax.broadcasted_iota(jnp.int32, (B, NBr), 1) == ib_r
                ).astype(jnp.bfloat16)
        feats_s = _dot_oh(oh_s, ts_ref[...])
        feats_r = _dot_oh(oh_r, tr_ref[...])
        w = [r[...] for r in w_refs]
        h = _edge_mlp_body(feats_s, feats_r, u_ref[0, 0], w, ds_pos, dr_pos)
        if scatter:
            ob = io_ref[0]
            oh_o = (lax.broadcasted_iota(jnp.int32, (NB_out, B), 0) == ob
                    ).astype(jnp.bfloat16)
            contrib = _dot_oh(oh_o, h)

            @pl.when(pl.program_id(0) == 0)
            def _():
                out_ref[...] = contrib

            @pl.when(pl.program_id(0) != 0)
            def _():
                out_ref[...] = out_ref[...] + contrib
        else:
            out_ref[...] = h[:, 0:D_out // 2]
            refs[k + 4 + nw][...] = h[:, D_out // 2:]

    in_specs = [pl.BlockSpec((1, B, 1), lambda i: (i, 0, 0)),
                pl.BlockSpec((1, B, 1), lambda i: (i, 0, 0))]
    args = [idx_s.reshape(NBLK, B, 1), idx_r.reshape(NBLK, B, 1)]
    if scatter:
        in_specs.append(pl.BlockSpec((1, 1, B), lambda i: (i, 0, 0)))
        args.append(idx_o.reshape(NBLK, 1, B))
    for a in (table_s, table_r, u, *wts):
        in_specs.append(pl.BlockSpec(a.shape, lambda i: (0, 0)))
        args.append(a)
    if scatter:
        out_shape = jax.ShapeDtypeStruct((NB_out, D_out), jnp.float32)
        out_spec = pl.BlockSpec((NB_out, D_out), lambda i: (0, 0))
    else:
        half = jax.ShapeDtypeStruct((E_pad, D_out // 2), jnp.float32)
        out_shape = [half, half]
        out_spec = [pl.BlockSpec((B, D_out // 2), lambda i: (i, 0))] * 2
    return pl.pallas_call(kern, grid=(NBLK,), in_specs=in_specs,
                          out_specs=out_spec, out_shape=out_shape)(*args)


def _edge_rows(rows_s, rows_r, u, wts, ds_pos, dr_pos, B, D_out):
    E_pad = rows_s.shape[0]
    NBLK = E_pad // B
    nw = len(wts)

    def kern(*refs):
        rs_ref, rr_ref, u_ref = refs[0], refs[1], refs[2]
        w = [r[...] for r in refs[3:3 + nw]]
        h = _edge_mlp_body(rs_ref[...], rr_ref[...], u_ref[0, 0],
                           w, ds_pos, dr_pos)
        refs[3 + nw][...] = h[:, 0:D_out // 2]
        refs[4 + nw][...] = h[:, D_out // 2:]

    in_specs = [pl.BlockSpec((B, rows_s.shape[1]), lambda i: (i, 0)),
                pl.BlockSpec((B, rows_r.shape[1]), lambda i: (i, 0))]
    args = [rows_s, rows_r]
    for a in (u, *wts):
        in_specs.append(pl.BlockSpec(a.shape, lambda i: (0, 0)))
        args.append(a)
    half = jax.ShapeDtypeStruct((E_pad, D_out // 2), jnp.float32)
    return pl.pallas_call(
        kern, grid=(NBLK,), in_specs=in_specs,
        out_specs=[pl.BlockSpec((B, D_out // 2), lambda i: (i, 0))] * 2,
        out_shape=[half, half])(*args)


def _node_stage(table, parts, u, lyrs, dv, da, B, final=None):
    N, Dtab = table.shape
    Dagg = parts[0].shape[1]
    W1 = lyrs[0]["W"]
    H1 = W1.shape[0]
    wts = [_pad2(W1[:, 0:dv].T, Dtab, H1),
           _pad2(W1[:, dv:dv + da].T, Dagg, H1),
           lyrs[0]["b"][None, :],
           W1[:, -1][None, :]]
    for lyr in lyrs[1:]:
        wts += [lyr["W"].T, lyr["b"][None, :]]
    D_out = lyrs[-1]["W"].shape[0]
    if final is not None:
        wts += [_pad2(final["W"].T, final["W"].shape[1], 8),
                _pad2(final["b"][None, :], 1, 8)]
        D_out = 8
    nw = len(wts)
    nparts = len(parts)
    NBLK = N // B

    def kern(*refs):
        t_ref = refs[0]
        agg = refs[1][...]
        for j in range(2, 1 + nparts):
            agg = agg + refs[j][...]
        u_ref = refs[1 + nparts]
        w = [r[...] for r in refs[2 + nparts:2 + nparts + nw]]
        out_ref = refs[2 + nparts + nw]
        h = (_dot(t_ref[...], w[0]) + _dot(agg, w[1]) + w[2]
             + _b(u_ref[0, 0]) * _b(w[3]))
        h = jnp.maximum(h, 0.0)
        nl = len(lyrs) - 1
        k = 4
        for _ in range(nl):
            h = jnp.maximum(_dot(h, w[k]) + w[k + 1], 0.0)
            k += 2
        if final is not None:
            h = _dot(h, w[k]) + w[k + 1]
        out_ref[...] = h

    in_specs = [pl.BlockSpec((B, Dtab), lambda i: (i, 0))]
    args = [table]
    for p in parts:
        in_specs.append(pl.BlockSpec((B, Dagg), lambda i: (i, 0)))
        args.append(p)
    for a in (u, *wts):
        in_specs.append(pl.BlockSpec(a.shape, lambda i: (0, 0)))
        args.append(a)
    return pl.pallas_call(
        kern, grid=(NBLK,), in_specs=in_specs,
        out_specs=pl.BlockSpec((B, D_out), lambda i: (i, 0)),
        out_shape=jax.ShapeDtypeStruct((N, D_out), jnp.float32))(*args)


@functools.cache
def _sc_mesh():
    return plsc.VectorSubcoreMesh(core_axis_name="c", subcore_axis_name="s")


def _sc_gather_pair(table, idx_s, idx_r, CH):
    E_pad = idx_s.shape[0]
    N, D = table.shape
    per_w = E_pad // 32
    nch = per_w // CH

    @functools.partial(
        pl.kernel,
        out_type=[jax.ShapeDtypeStruct((E_pad, D), jnp.float32),
                  jax.ShapeDtypeStruct((E_pad, D), jnp.float32)],
        mesh=_sc_mesh(),
        compiler_params=pltpu.CompilerParams(use_tc_tiling_on_sc=False),
        scratch_types=[pltpu.VMEM((2, CH), jnp.int32),
                       pltpu.VMEM((2, CH), jnp.int32),
                       pltpu.VMEM((2, CH, D), jnp.float32),
                       pltpu.VMEM((2, CH, D), jnp.float32)]
        + [pltpu.SemaphoreType.DMA] * 6)
    def k(tab, isrc, irsc, outs, outr, iv_s, iv_r, rv_s, rv_r,
          si, sg, sw, si1, sg1, sw1):
        wid = lax.axis_index("s") * 2 + lax.axis_index("c")
        base = wid * per_w
        sem_i, sem_g, sem_w = (si, si1), (sg, sg1), (sw, sw1)

        def start_idx(j, b):
            off = base + j * CH
            return (pltpu.async_copy(isrc.at[pl.ds(off, CH)], iv_s.at[b],
                                     sem_i[b]),
                    pltpu.async_copy(irsc.at[pl.ds(off, CH)], iv_r.at[b],
                                     sem_i[b]))

        idx_d = [None, None]
        wb_d = [None, None]
        idx_d[0] = start_idx(0, 0)
        for j in range(nch):
            b = j & 1
            nb = 1 - b
            if j + 1 < nch:
                idx_d[nb] = start_idx(j + 1, nb)
            for c in idx_d[b]:
                c.wait()
            if wb_d[b] is not None:
                for c in wb_d[b]:
                    c.wait()
            cs = pltpu.async_copy(tab.at[iv_s.at[b]], rv_s.at[b], sem_g[b])
            cr = pltpu.async_copy(tab.at[iv_r.at[b]], rv_r.at[b], sem_g[b])
            cs.wait()
            cr.wait()
            off = base + j * CH
            wb_d[b] = (pltpu.async_copy(rv_s.at[b], outs.at[pl.ds(off, CH)],
                                        sem_w[b]),
                       pltpu.async_copy(rv_r.at[b], outr.at[pl.ds(off, CH)],
                                        sem_w[b]))
        for b in (0, 1):
            if wb_d[b] is not None:
                for c in wb_d[b]:
                    c.wait()

    return k(table, idx_s, idx_r)


def _sc_scatter_add(rows_list, idx_list, N, D, CH):
    stripe = N // 16
    per_w = [r.shape[0] // 32 for r in rows_list]
    nch = [p // CH for p in per_w]
    zeros = jnp.zeros((stripe, D), jnp.float32)

    seq = [(a, i) for a in range(len(rows_list)) for i in range(nch[a])]

    @functools.partial(
        pl.kernel,
        out_type=jax.ShapeDtypeStruct((2 * N, D), jnp.float32),
        mesh=_sc_mesh(),
        compiler_params=pltpu.CompilerParams(use_tc_tiling_on_sc=False),
        scratch_types=[pltpu.VMEM((2, CH), jnp.int32),
                       pltpu.VMEM((2, CH, D), jnp.float32),
                       pltpu.VMEM_SHARED((N, D), jnp.float32)]
        + [pltpu.SemaphoreType.DMA] * 4)
    def k(*refs):
        na = len(rows_list)
        rows = refs[0:na]
        idxs = refs[na:2 * na]
        zref = refs[2 * na]
        out = refs[2 * na + 1]
        iv, rv, acc, sl0, sl1, ss0, ss1 = refs[2 * na + 2:2 * na + 9]
        sem_l, sem_s = (sl0, sl1), (ss0, ss1)
        c = lax.axis_index("c")
        s = lax.axis_index("s")
        pltpu.sync_copy(zref, acc.at[pl.ds(s * stripe, stripe)])
        plsc.subcore_barrier()
        wid = s * 2 + c

        def start_load(j, b):
            a, i = seq[j]
            off = wid * per_w[a] + i * CH
            return (pltpu.async_copy(idxs[a].at[pl.ds(off, CH)], iv.at[b],
                                     sem_l[b]),
                    pltpu.async_copy(rows[a].at[pl.ds(off, CH)], rv.at[b],
                                     sem_l[b]))

        ld_d = [None, None]
        sc_d = [None, None]
        ld_d[0] = start_load(0, 0)
        for j in range(len(seq)):
            b = j & 1
            nb = 1 - b
            if j + 1 < len(seq):
                if sc_d[nb] is not None:
                    sc_d[nb].wait()
                    sc_d[nb] = None
                ld_d[nb] = start_load(j + 1, nb)
            for cpy in ld_d[b]:
                cpy.wait()
            if sc_d[b] is not None:
                sc_d[b].wait()
            sc_d[b] = pltpu.async_copy(rv.at[b], acc.at[iv.at[b]], sem_s[b],
                                       add=True)
        for b in (0, 1):
            if sc_d[b] is not None:
                sc_d[b].wait()
        plsc.subcore_barrier()
        pltpu.sync_copy(acc.at[pl.ds(s * stripe, stripe)],
                        out.at[pl.ds(c * N + s * stripe, stripe)])

    out = k(*rows_list, *idx_list, zeros)
    return out[0:N] + out[N:]


_NP_COLS = (0, 3, 4)


def kernel(V, R_s, R_r, assignments_0, assignments_1, V_supers_0, V_supers_1,
           super_graphs_0, super_graphs_1, dt, params):
    V = V[0]
    R_s = R_s[0].astype(jnp.int32)
    R_r = R_r[0].astype(jnp.int32)
    a0 = assignments_0[0].astype(jnp.int32)
    a1 = assignments_1[0].astype(jnp.int32)
    Vs0 = V_supers_0[0]
    Vs1 = V_supers_1[0]
    sg0 = super_graphs_0[0].astype(jnp.int32)
    sg1 = super_graphs_1[0].astype(jnp.int32)
    u = dt.reshape(1, 1).astype(jnp.float32)
    npc = jnp.array(_NP_COLS)

    Vnp, Vpos = V[:, npc], V[:, 1:3]
    Vs0np, Vs0pos = Vs0[:, npc], Vs0[:, 1:3]
    Vs1np, Vs1pos = Vs1[:, npc], Vs1[:, 1:3]
    Vtab = _pad2(jnp.concatenate([Vnp, Vpos], 1), 10240, 16)
    Vs0tab = _pad2(jnp.concatenate([Vs0np, Vs0pos], 1), 1024, 16)
    Vs1tab = _pad2(jnp.concatenate([Vs1np, Vs1pos], 1), 128, 16)

    w1 = _prep_edge_w(params["edge_to_super"], 3, 3, 16, 16, 112)
    agg1 = _edge_onehot(_padi(a0[:, 1], 10240, 0), _padi(a0[:, 0], 10240, 0),
                        Vtab[0:1024], Vs0tab, u, w1, 3, 3, B=1024, D_out=112,
                        idx_o=_padi(a0[:, 0], 10240, 1023), NB_out=1024)
    VLtab = _pad2(jnp.concatenate([Vs0np, agg1[0:1000, 0:100], Vs0pos], 1),
                  1024, 112)

    w2 = _prep_edge_w(params["edge_to_upper"], 103, 3, 112, 16, 112)
    agg2 = _edge_onehot(_padi(a1[:, 1], 1024, 0), _padi(a1[:, 0], 1024, 0),
                        VLtab[0:128], Vs1tab, u, w2, 103, 3, B=512, D_out=112,
                        idx_o=_padi(a1[:, 0], 1024, 127), NB_out=128)
    Vtoptab = _pad2(jnp.concatenate([Vs1np, agg2[0:100, 0:100], Vs1pos], 1),
                    128, 112)

    w3 = _prep_edge_w(params["super_edge"], 103, 103, 112, 112, 160)
    agg3 = _edge_onehot(_padi(sg1[:, 0], 2048, 0), _padi(sg1[:, 1], 2048, 0),
                        Vtoptab, Vtoptab, u, w3, 103, 103, B=512, D_out=160,
                        idx_o=_padi(sg1[:, 1], 2048, 127), NB_out=128)
    vnew3 = _node_stage(Vtoptab, [agg3], u, params["super_node"], 103, 150,
                        B=128)
    Vuptab = _pad2(jnp.concatenate([Vs1np, vnew3[0:100, 0:100], Vs1pos], 1),
                   128, 112)

    w4a = _prep_edge_w(params["edge_from_upper"], 103, 103, 112, 112, 160)
    p4a = _edge_onehot(_padi(a1[:, 0], 1024, 0), _padi(a1[:, 1], 1024, 0),
                       Vuptab, VLtab[0:128], u, w4a, 103, 103, B=512,
                       D_out=160, idx_o=_padi(a1[:, 1], 1024, 1023),
                       NB_out=1024)
    p4b = _edge_onehot(_padi(sg0[:, 0], 16384, 0), _padi(sg0[:, 1], 16384, 0),
                       VLtab, VLtab, u, w3, 103, 103, B=2048, D_out=160,
                       idx_o=_padi(sg0[:, 1], 16384, 1023), NB_out=1024)
    vnew4 = _node_stage(VLtab, [p4a, p4b], u, params["super_node"], 103, 150,
                        B=512)
    Vup2tab = _pad2(jnp.concatenate([Vs0np, vnew4[0:1000, 0:100], Vs0pos], 1),
                    1024, 112)

    gs, gr = _sc_gather_pair(Vtab, _padi(R_s, 163840, 0),
                             _padi(R_r, 163840, 0), CH=1280)
    w5e = _prep_edge_w(params["edge"], 3, 3, 16, 16, 160)
    en_lo, en_hi = _edge_rows(gs, gr, u, w5e, 3, 3, B=2048, D_out=160)
    w5a = _prep_edge_w(params["edge_from_super"], 103, 3, 112, 16, 160)
    ui_lo, ui_hi = _edge_onehot(_padi(a0[:, 0], 10240, 0),
                                _padi(a0[:, 1], 10240, 0),
                                Vup2tab, Vtab[0:1024], u, w5a, 103, 3, B=2048,
                                D_out=160)
    idx1 = _padi(R_r, 163840, 10200)
    idx2 = _padi(a0[:, 1], 10240, 10200)
    agg_lo = _sc_scatter_add([en_lo, ui_lo], [idx1, idx2], N=10240, D=80,
                             CH=320)
    agg_hi = _sc_scatter_add([en_hi, ui_hi], [idx1, idx2], N=10240, D=80,
                             CH=320)
    agg5 = jnp.concatenate([agg_lo, agg_hi], 1)
    out5 = _node_stage(Vtab, [agg5], u,
                       params["node"], 3, 150, B=512, final=params["linear"])
    return out5[0:10000, 0:4][None]


# speedup vs baseline: 1.1351x; 1.0310x over previous
import functools

import jax
import jax.numpy as jnp
from jax import lax
from jax.experimental import pallas as pl
from jax.experimental.pallas import tpu as pltpu
from jax.experimental.pallas import tpu_sc as plsc

_BOX = 6.0


def _dot1(a, b):
    return lax.dot_general(a, b, (((1,), (0,)), ((), ())),
                           preferred_element_type=jnp.float32)


def _split(x):
    xh = x.astype(jnp.bfloat16)
    return xh, (x - xh.astype(jnp.float32)).astype(jnp.bfloat16)


def _dot(a, b):
    return _dot1(a.astype(jnp.bfloat16), b.astype(jnp.bfloat16))


def _b(x):
    return x.astype(jnp.bfloat16).astype(jnp.float32)


def _dot_oh(oh, b):
    bh, bl = _split(b)
    return _dot1(oh, bh) + _dot1(oh, bl)


def _pad2(x, r, c):
    return jnp.pad(x, ((0, r - x.shape[0]), (0, c - x.shape[1])))


def _padi(x, n, fill):
    return jnp.pad(x, (0, n - x.shape[0]), constant_values=fill).astype(jnp.int32)


def _prep_edge_w(lyrs, ds_f, dr_f, Dts, Dtr, D_out):
    W1 = lyrs[0]["W"]
    H1 = W1.shape[0]
    out = [_pad2(W1[:, 0:ds_f].T, Dts, H1),
           _pad2(W1[:, ds_f:ds_f + dr_f].T, Dtr, H1),
           W1[:, ds_f + dr_f][None, :],
           W1[:, ds_f + dr_f + 1][None, :],
           lyrs[0]["b"][None, :],
           W1[:, -1][None, :]]
    for i, lyr in enumerate(lyrs[1:]):
        WT, bb = lyr["W"].T, lyr["b"][None, :]
        if i == len(lyrs) - 2:
            WT, bb = _pad2(WT, WT.shape[0], D_out), _pad2(bb, 1, D_out)
        out += [WT, bb]
    return out


def _edge_mlp_body(feats_s, feats_r, u, w, ds_pos, dr_pos):
    W1sT, W1rT, p0, p1, b1, w1u = w[:6]
    rel = feats_s[:, ds_pos:ds_pos + 2] - feats_r[:, dr_pos:dr_pos + 2]
    rel = jnp.where(rel > _BOX / 2, rel - _BOX, rel)
    rel = jnp.where(rel <= -_BOX / 2, rel + _BOX, rel)
    h = _dot(feats_s, W1sT) + _dot(feats_r, W1rT)
    relb = _b(rel)
    h = (h + relb[:, 0:1] * _b(p0) + relb[:, 1:2] * _b(p1) + b1
         + _b(u) * _b(w1u))
    h = jnp.maximum(h, 0.0)
    for j in range(6, len(w), 2):
        h = jnp.maximum(_dot(h, w[j]) + w[j + 1], 0.0)
    return h


def _edge_onehot(idx_s, idx_r, table_s, table_r, u, wts, ds_pos, dr_pos, B,
                 D_out, idx_o=None, NB_out=None):
    E_pad = idx_s.shape[0]
    NBLK = E_pad // B
    NBs, NBr = table_s.shape[0], table_r.shape[0]
    nw = len(wts)
    scatter = idx_o is not None

    def kern(*refs):
        is_ref, ir_ref = refs[0], refs[1]
        k = 2
        if scatter:
            io_ref = refs[2]
            k = 3
        ts_ref, tr_ref, u_ref = refs[k:k + 3]
        w_refs = refs[k + 3:k + 3 + nw]
        out_ref = refs[k + 3 + nw]
        ib_s = is_ref[0]
        ib_r = ir_ref[0]
        oh_s = (lax.broadcasted_iota(jnp.int32, (B, NBs), 1) == ib_s
                ).astype(jnp.bfloat16)
        oh_r = (lax.broadcasted_iota(jnp.int32, (B, NBr), 1) == ib_r
                ).astype(jnp.bfloat16)
        feats_s = _dot_oh(oh_s, ts_ref[...])
        feats_r = _dot_oh(oh_r, tr_ref[...])
        w = [r[...] for r in w_refs]
        h = _edge_mlp_body(feats_s, feats_r, u_ref[0, 0], w, ds_pos, dr_pos)
        if scatter:
            ob = io_ref[0]
            oh_o = (lax.broadcasted_iota(jnp.int32, (NB_out, B), 0) == ob
                    ).astype(jnp.bfloat16)
            contrib = _dot_oh(oh_o, h)

            @pl.when(pl.program_id(0) == 0)
            def _():
                out_ref[...] = contrib

            @pl.when(pl.program_id(0) != 0)
            def _():
                out_ref[...] = out_ref[...] + contrib
        else:
            out_ref[...] = h[:, 0:D_out // 2]
            refs[k + 4 + nw][...] = h[:, D_out // 2:]

    in_specs = [pl.BlockSpec((1, B, 1), lambda i: (i, 0, 0)),
                pl.BlockSpec((1, B, 1), lambda i: (i, 0, 0))]
    args = [idx_s.reshape(NBLK, B, 1), idx_r.reshape(NBLK, B, 1)]
    if scatter:
        in_specs.append(pl.BlockSpec((1, 1, B), lambda i: (i, 0, 0)))
        args.append(idx_o.reshape(NBLK, 1, B))
    for a in (table_s, table_r, u, *wts):
        in_specs.append(pl.BlockSpec(a.shape, lambda i: (0, 0)))
        args.append(a)
    if scatter:
        out_shape = jax.ShapeDtypeStruct((NB_out, D_out), jnp.float32)
        out_spec = pl.BlockSpec((NB_out, D_out), lambda i: (0, 0))
    else:
        half = jax.ShapeDtypeStruct((E_pad, D_out // 2), jnp.float32)
        out_shape = [half, half]
        out_spec = [pl.BlockSpec((B, D_out // 2), lambda i: (i, 0))] * 2
    return pl.pallas_call(kern, grid=(NBLK,), in_specs=in_specs,
                          out_specs=out_spec, out_shape=out_shape)(*args)


def _edge_rows(rows_s, rows_r, u, wts, ds_pos, dr_pos, B, D_out):
    E_pad = rows_s.shape[0]
    NBLK = E_pad // B
    nw = len(wts)

    def kern(*refs):
        rs_ref, rr_ref, u_ref = refs[0], refs[1], refs[2]
        w = [r[...] for r in refs[3:3 + nw]]
        h = _edge_mlp_body(rs_ref[...], rr_ref[...], u_ref[0, 0],
                           w, ds_pos, dr_pos)
        refs[3 + nw][...] = h[:, 0:D_out // 2]
        refs[4 + nw][...] = h[:, D_out // 2:]

    in_specs = [pl.BlockSpec((B, rows_s.shape[1]), lambda i: (i, 0)),
                pl.BlockSpec((B, rows_r.shape[1]), lambda i: (i, 0))]
    args = [rows_s, rows_r]
    for a in (u, *wts):
        in_specs.append(pl.BlockSpec(a.shape, lambda i: (0, 0)))
        args.append(a)
    half = jax.ShapeDtypeStruct((E_pad, D_out // 2), jnp.float32)
    return pl.pallas_call(
        kern, grid=(NBLK,), in_specs=in_specs,
        out_specs=[pl.BlockSpec((B, D_out // 2), lambda i: (i, 0))] * 2,
        out_shape=[half, half])(*args)


def _node_stage(table, parts, u, lyrs, dv, da, B, final=None):
    N, Dtab = table.shape
    Dagg = parts[0].shape[1]
    W1 = lyrs[0]["W"]
    H1 = W1.shape[0]
    wts = [_pad2(W1[:, 0:dv].T, Dtab, H1),
           _pad2(W1[:, dv:dv + da].T, Dagg, H1),
           lyrs[0]["b"][None, :],
           W1[:, -1][None, :]]
    for lyr in lyrs[1:]:
        wts += [lyr["W"].T, lyr["b"][None, :]]
    D_out = lyrs[-1]["W"].shape[0]
    if final is not None:
        wts += [_pad2(final["W"].T, final["W"].shape[1], 8),
                _pad2(final["b"][None, :], 1, 8)]
        D_out = 8
    nw = len(wts)
    nparts = len(parts)
    NBLK = N // B

    def kern(*refs):
        t_ref = refs[0]
        agg = refs[1][...]
        for j in range(2, 1 + nparts):
            agg = agg + refs[j][...]
        u_ref = refs[1 + nparts]
        w = [r[...] for r in refs[2 + nparts:2 + nparts + nw]]
        out_ref = refs[2 + nparts + nw]
        h = (_dot(t_ref[...], w[0]) + _dot(agg, w[1]) + w[2]
             + _b(u_ref[0, 0]) * _b(w[3]))
        h = jnp.maximum(h, 0.0)
        nl = len(lyrs) - 1
        k = 4
        for _ in range(nl):
            h = jnp.maximum(_dot(h, w[k]) + w[k + 1], 0.0)
            k += 2
        if final is not None:
            h = _dot(h, w[k]) + w[k + 1]
        out_ref[...] = h

    in_specs = [pl.BlockSpec((B, Dtab), lambda i: (i, 0))]
    args = [table]
    for p in parts:
        in_specs.append(pl.BlockSpec((B, Dagg), lambda i: (i, 0)))
        args.append(p)
    for a in (u, *wts):
        in_specs.append(pl.BlockSpec(a.shape, lambda i: (0, 0)))
        args.append(a)
    return pl.pallas_call(
        kern, grid=(NBLK,), in_specs=in_specs,
        out_specs=pl.BlockSpec((B, D_out), lambda i: (i, 0)),
        out_shape=jax.ShapeDtypeStruct((N, D_out), jnp.float32))(*args)


@functools.cache
def _sc_mesh():
    return plsc.VectorSubcoreMesh(core_axis_name="c", subcore_axis_name="s")


def _sc_gather_pair(table, idx_s, idx_r, CH):
    E_pad = idx_s.shape[0]
    N, D = table.shape
    per_w = E_pad // 32
    nch = per_w // CH

    @functools.partial(
        pl.kernel,
        out_type=[jax.ShapeDtypeStruct((E_pad, D), jnp.float32),
                  jax.ShapeDtypeStruct((E_pad, D), jnp.float32)],
        mesh=_sc_mesh(),
        compiler_params=pltpu.CompilerParams(use_tc_tiling_on_sc=False),
        scratch_types=[pltpu.VMEM((2, CH), jnp.int32),
                       pltpu.VMEM((2, CH), jnp.int32),
                       pltpu.VMEM((2, CH, D), jnp.float32),
                       pltpu.VMEM((2, CH, D), jnp.float32)]
        + [pltpu.SemaphoreType.DMA] * 6)
    def k(tab, isrc, irsc, outs, outr, iv_s, iv_r, rv_s, rv_r,
          si, sg, sw, si1, sg1, sw1):
        wid = lax.axis_index("s") * 2 + lax.axis_index("c")
        base = wid * per_w
        sem_i, sem_g, sem_w = (si, si1), (sg, sg1), (sw, sw1)

        def start_idx(j, b):
            off = base + j * CH
            return (pltpu.async_copy(isrc.at[pl.ds(off, CH)], iv_s.at[b],
                                     sem_i[b]),
                    pltpu.async_copy(irsc.at[pl.ds(off, CH)], iv_r.at[b],
                                     sem_i[b]))

        idx_d = [None, None]
        wb_d = [None, None]
        idx_d[0] = start_idx(0, 0)
        for j in range(nch):
            b = j & 1
            nb = 1 - b
            if j + 1 < nch:
                idx_d[nb] = start_idx(j + 1, nb)
            for c in idx_d[b]:
                c.wait()
            if wb_d[b] is not None:
                for c in wb_d[b]:
                    c.wait()
            cs = pltpu.async_copy(tab.at[iv_s.at[b]], rv_s.at[b], sem_g[b])
            cr = pltpu.async_copy(tab.at[iv_r.at[b]], rv_r.at[b], sem_g[b])
            cs.wait()
            cr.wait()
            off = base + j * CH
            wb_d[b] = (pltpu.async_copy(rv_s.at[b], outs.at[pl.ds(off, CH)],
                                        sem_w[b]),
                       pltpu.async_copy(rv_r.at[b], outr.at[pl.ds(off, CH)],
                                        sem_w[b]))
        for b in (0, 1):
            if wb_d[b] is not None:
                for c in wb_d[b]:
                    c.wait()

    return k(table, idx_s, idx_r)


def _sc_scatter_add(rows_list, idx_list, N, D, CH):
    stripe = N // 16
    per_w = [r.shape[0] // 32 for r in rows_list]
    nch = [p // CH for p in per_w]
    zeros = jnp.zeros((stripe, D), jnp.float32)

    seq = [(a, i) for a in range(len(rows_list)) for i in range(nch[a])]

    @functools.partial(
        pl.kernel,
        out_type=jax.ShapeDtypeStruct((2 * N, D), jnp.float32),
        mesh=_sc_mesh(),
        compiler_params=pltpu.CompilerParams(use_tc_tiling_on_sc=False),
        scratch_types=[pltpu.VMEM((2, CH), jnp.int32),
                       pltpu.VMEM((2, CH, D), jnp.float32),
                       pltpu.VMEM_SHARED((N, D), jnp.float32)]
        + [pltpu.SemaphoreType.DMA] * 4)
    def k(*refs):
        na = len(rows_list)
        rows = refs[0:na]
        idxs = refs[na:2 * na]
        zref = refs[2 * na]
        out = refs[2 * na + 1]
        iv, rv, acc, sl0, sl1, ss0, ss1 = refs[2 * na + 2:2 * na + 9]
        sem_l, sem_s = (sl0, sl1), (ss0, ss1)
        c = lax.axis_index("c")
        s = lax.axis_index("s")
        pltpu.sync_copy(zref, acc.at[pl.ds(s * stripe, stripe)])
        plsc.subcore_barrier()
        wid = s * 2 + c

        def start_load(j, b):
            a, i = seq[j]
            off = wid * per_w[a] + i * CH
            return (pltpu.async_copy(idxs[a].at[pl.ds(off, CH)], iv.at[b],
                                     sem_l[b]),
                    pltpu.async_copy(rows[a].at[pl.ds(off, CH)], rv.at[b],
                                     sem_l[b]))

        ld_d = [None, None]
        sc_d = [None, None]
        ld_d[0] = start_load(0, 0)
        for j in range(len(seq)):
            b = j & 1
            nb = 1 - b
            if j + 1 < len(seq):
                if sc_d[nb] is not None:
                    sc_d[nb].wait()
                    sc_d[nb] = None
                ld_d[nb] = start_load(j + 1, nb)
            for cpy in ld_d[b]:
                cpy.wait()
            if sc_d[b] is not None:
                sc_d[b].wait()
            sc_d[b] = pltpu.async_copy(rv.at[b], acc.at[iv.at[b]], sem_s[b],
                                       add=True)
        for b in (0, 1):
            if sc_d[b] is not None:
                sc_d[b].wait()
        plsc.subcore_barrier()
        pltpu.sync_copy(acc.at[pl.ds(s * stripe, stripe)],
                        out.at[pl.ds(c * N + s * stripe, stripe)])

    out = k(*rows_list, *idx_list, zeros)
    return out[0:N] + out[N:]


_NP_COLS = (0, 3, 4)


def kernel(V, R_s, R_r, assignments_0, assignments_1, V_supers_0, V_supers_1,
           super_graphs_0, super_graphs_1, dt, params):
    V = V[0]
    R_s = R_s[0].astype(jnp.int32)
    R_r = R_r[0].astype(jnp.int32)
    a0 = assignments_0[0].astype(jnp.int32)
    a1 = assignments_1[0].astype(jnp.int32)
    Vs0 = V_supers_0[0]
    Vs1 = V_supers_1[0]
    sg0 = super_graphs_0[0].astype(jnp.int32)
    sg1 = super_graphs_1[0].astype(jnp.int32)
    u = dt.reshape(1, 1).astype(jnp.float32)
    npc = jnp.array(_NP_COLS)

    Vnp, Vpos = V[:, npc], V[:, 1:3]
    Vs0np, Vs0pos = Vs0[:, npc], Vs0[:, 1:3]
    Vs1np, Vs1pos = Vs1[:, npc], Vs1[:, 1:3]
    Vtab = _pad2(jnp.concatenate([Vnp, Vpos], 1), 10240, 16)
    Vs0tab = _pad2(jnp.concatenate([Vs0np, Vs0pos], 1), 1024, 16)
    Vs1tab = _pad2(jnp.concatenate([Vs1np, Vs1pos], 1), 128, 16)

    w1 = _prep_edge_w(params["edge_to_super"], 3, 3, 16, 16, 112)
    agg1 = _edge_onehot(_padi(a0[:, 1], 10240, 0), _padi(a0[:, 0], 10240, 0),
                        Vtab[0:1024], Vs0tab, u, w1, 3, 3, B=1024, D_out=112,
                        idx_o=_padi(a0[:, 0], 10240, 1023), NB_out=1024)
    VLtab = _pad2(jnp.concatenate([Vs0np, agg1[0:1000, 0:100], Vs0pos], 1),
                  1024, 112)

    w2 = _prep_edge_w(params["edge_to_upper"], 103, 3, 112, 16, 112)
    agg2 = _edge_onehot(_padi(a1[:, 1], 1024, 0), _padi(a1[:, 0], 1024, 0),
                        VLtab[0:128], Vs1tab, u, w2, 103, 3, B=512, D_out=112,
                        idx_o=_padi(a1[:, 0], 1024, 127), NB_out=128)
    Vtoptab = _pad2(jnp.concatenate([Vs1np, agg2[0:100, 0:100], Vs1pos], 1),
                    128, 112)

    w3 = _prep_edge_w(params["super_edge"], 103, 103, 112, 112, 160)
    agg3 = _edge_onehot(_padi(sg1[:, 0], 2048, 0), _padi(sg1[:, 1], 2048, 0),
                        Vtoptab, Vtoptab, u, w3, 103, 103, B=512, D_out=160,
                        idx_o=_padi(sg1[:, 1], 2048, 127), NB_out=128)
    vnew3 = _node_stage(Vtoptab, [agg3], u, params["super_node"], 103, 150,
                        B=128)
    Vuptab = _pad2(jnp.concatenate([Vs1np, vnew3[0:100, 0:100], Vs1pos], 1),
                   128, 112)

    w4a = _prep_edge_w(params["edge_from_upper"], 103, 103, 112, 112, 160)
    p4a = _edge_onehot(_padi(a1[:, 0], 1024, 0), _padi(a1[:, 1], 1024, 0),
                       Vuptab, VLtab[0:128], u, w4a, 103, 103, B=512,
                       D_out=160, idx_o=_padi(a1[:, 1], 1024, 1023),
                       NB_out=1024)
    p4b = _edge_onehot(_padi(sg0[:, 0], 16384, 0), _padi(sg0[:, 1], 16384, 0),
                       VLtab, VLtab, u, w3, 103, 103, B=2048, D_out=160,
                       idx_o=_padi(sg0[:, 1], 16384, 1023), NB_out=1024)
    vnew4 = _node_stage(VLtab, [p4a, p4b], u, params["super_node"], 103, 150,
                        B=512)
    Vup2tab = _pad2(jnp.concatenate([Vs0np, vnew4[0:1000, 0:100], Vs0pos], 1),
                    1024, 112)

    gs, gr = _sc_gather_pair(Vtab, _padi(R_s, 163840, 0),
                             _padi(R_r, 163840, 0), CH=1280)
    w5e = _prep_edge_w(params["edge"], 3, 3, 16, 16, 160)
    en_lo, en_hi = _edge_rows(gs, gr, u, w5e, 3, 3, B=4096, D_out=160)
    w5a = _prep_edge_w(params["edge_from_super"], 103, 3, 112, 16, 160)
    ui_lo, ui_hi = _edge_onehot(_padi(a0[:, 0], 10240, 0),
                                _padi(a0[:, 1], 10240, 0),
                                Vup2tab, Vtab[0:1024], u, w5a, 103, 3, B=5120,
                                D_out=160)
    idx1 = _padi(R_r, 163840, 10200)
    idx2 = _padi(a0[:, 1], 10240, 10200)
    agg_lo = _sc_scatter_add([en_lo, ui_lo], [idx1, idx2], N=10240, D=80,
                             CH=320)
    agg_hi = _sc_scatter_add([en_hi, ui_hi], [idx1, idx2], N=10240, D=80,
                             CH=320)
    agg5 = jnp.concatenate([agg_lo, agg_hi], 1)
    out5 = _node_stage(Vtab, [agg5], u,
                       params["node"], 3, 150, B=512, final=params["linear"])
    return out5[0:10000, 0:4][None]


# speedup vs baseline: 1.1388x; 1.0033x over previous
import functools

import jax
import jax.numpy as jnp
from jax import lax
from jax.experimental import pallas as pl
from jax.experimental.pallas import tpu as pltpu
from jax.experimental.pallas import tpu_sc as plsc

_BOX = 6.0


def _dot1(a, b):
    return lax.dot_general(a, b, (((1,), (0,)), ((), ())),
                           preferred_element_type=jnp.float32)


def _split(x):
    xh = x.astype(jnp.bfloat16)
    return xh, (x - xh.astype(jnp.float32)).astype(jnp.bfloat16)


def _dot(a, b):
    return _dot1(a.astype(jnp.bfloat16), b.astype(jnp.bfloat16))


def _b(x):
    return x.astype(jnp.bfloat16).astype(jnp.float32)


def _dot_oh(oh, b):
    bh, bl = _split(b)
    return _dot1(oh, bh) + _dot1(oh, bl)


def _pad2(x, r, c):
    return jnp.pad(x, ((0, r - x.shape[0]), (0, c - x.shape[1])))


def _padi(x, n, fill):
    return jnp.pad(x, (0, n - x.shape[0]), constant_values=fill).astype(jnp.int32)


def _prep_edge_w(lyrs, ds_f, dr_f, Dts, Dtr, D_out):
    W1 = lyrs[0]["W"]
    H1 = W1.shape[0]
    out = [_pad2(W1[:, 0:ds_f].T, Dts, H1),
           _pad2(W1[:, ds_f:ds_f + dr_f].T, Dtr, H1),
           W1[:, ds_f + dr_f][None, :],
           W1[:, ds_f + dr_f + 1][None, :],
           lyrs[0]["b"][None, :],
           W1[:, -1][None, :]]
    for i, lyr in enumerate(lyrs[1:]):
        WT, bb = lyr["W"].T, lyr["b"][None, :]
        if i == len(lyrs) - 2:
            WT, bb = _pad2(WT, WT.shape[0], D_out), _pad2(bb, 1, D_out)
        out += [WT, bb]
    return out


def _edge_mlp_body(feats_s, feats_r, u, w, ds_pos, dr_pos):
    W1sT, W1rT, p0, p1, b1, w1u = w[:6]
    rel = feats_s[:, ds_pos:ds_pos + 2] - feats_r[:, dr_pos:dr_pos + 2]
    rel = jnp.where(rel > _BOX / 2, rel - _BOX, rel)
    rel = jnp.where(rel <= -_BOX / 2, rel + _BOX, rel)
    h = _dot(feats_s, W1sT) + _dot(feats_r, W1rT)
    relb = _b(rel)
    h = (h + relb[:, 0:1] * _b(p0) + relb[:, 1:2] * _b(p1) + b1
         + _b(u) * _b(w1u))
    h = jnp.maximum(h, 0.0)
    for j in range(6, len(w), 2):
        h = jnp.maximum(_dot(h, w[j]) + w[j + 1], 0.0)
    return h


def _edge_onehot(idx_s, idx_r, table_s, table_r, u, wts, ds_pos, dr_pos, B,
                 D_out, idx_o=None, NB_out=None):
    E_pad = idx_s.shape[0]
    NBLK = E_pad // B
    NBs, NBr = table_s.shape[0], table_r.shape[0]
    nw = len(wts)
    scatter = idx_o is not None

    def kern(*refs):
        is_ref, ir_ref = refs[0], refs[1]
        k = 2
        if scatter:
            io_ref = refs[2]
            k = 3
        ts_ref, tr_ref, u_ref = refs[k:k + 3]
        w_refs = refs[k + 3:k + 3 + nw]
        out_ref = refs[k + 3 + nw]
        ib_s = is_ref[0]
        ib_r = ir_ref[0]
        oh_s = (lax.broadcasted_iota(jnp.int32, (B, NBs), 1) == ib_s
                ).astype(jnp.bfloat16)
        oh_r = (lax.broadcasted_iota(jnp.int32, (B, NBr), 1) == ib_r
                ).astype(jnp.bfloat16)
        feats_s = _dot_oh(oh_s, ts_ref[...])
        feats_r = _dot_oh(oh_r, tr_ref[...])
        w = [r[...] for r in w_refs]
        h = _edge_mlp_body(feats_s, feats_r, u_ref[0, 0], w, ds_pos, dr_pos)
        if scatter:
            ob = io_ref[0]
            oh_o = (lax.broadcasted_iota(jnp.int32, (NB_out, B), 0) == ob
                    ).astype(jnp.bfloat16)
            contrib = _dot_oh(oh_o, h)

            @pl.when(pl.program_id(0) == 0)
            def _():
                out_ref[...] = contrib

            @pl.when(pl.program_id(0) != 0)
            def _():
                out_ref[...] = out_ref[...] + contrib
        else:
            out_ref[...] = h[:, 0:D_out // 2]
            refs[k + 4 + nw][...] = h[:, D_out // 2:]

    in_specs = [pl.BlockSpec((1, B, 1), lambda i: (i, 0, 0)),
                pl.BlockSpec((1, B, 1), lambda i: (i, 0, 0))]
    args = [idx_s.reshape(NBLK, B, 1), idx_r.reshape(NBLK, B, 1)]
    if scatter:
        in_specs.append(pl.BlockSpec((1, 1, B), lambda i: (i, 0, 0)))
        args.append(idx_o.reshape(NBLK, 1, B))
    for a in (table_s, table_r, u, *wts):
        in_specs.append(pl.BlockSpec(a.shape, lambda i: (0, 0)))
        args.append(a)
    if scatter:
        out_shape = jax.ShapeDtypeStruct((NB_out, D_out), jnp.float32)
        out_spec = pl.BlockSpec((NB_out, D_out), lambda i: (0, 0))
    else:
        half = jax.ShapeDtypeStruct((E_pad, D_out // 2), jnp.float32)
        out_shape = [half, half]
        out_spec = [pl.BlockSpec((B, D_out // 2), lambda i: (i, 0))] * 2
    return pl.pallas_call(kern, grid=(NBLK,), in_specs=in_specs,
                          out_specs=out_spec, out_shape=out_shape)(*args)


def _edge_rows(rows_s, rows_r, u, wts, ds_pos, dr_pos, B, D_out):
    E_pad = rows_s.shape[0]
    NBLK = E_pad // B
    nw = len(wts)

    def kern(*refs):
        rs_ref, rr_ref, u_ref = refs[0], refs[1], refs[2]
        w = [r[...] for r in refs[3:3 + nw]]
        h = _edge_mlp_body(rs_ref[...], rr_ref[...], u_ref[0, 0],
                           w, ds_pos, dr_pos)
        refs[3 + nw][...] = h[:, 0:D_out // 2]
        refs[4 + nw][...] = h[:, D_out // 2:]

    in_specs = [pl.BlockSpec((B, rows_s.shape[1]), lambda i: (i, 0)),
                pl.BlockSpec((B, rows_r.shape[1]), lambda i: (i, 0))]
    args = [rows_s, rows_r]
    for a in (u, *wts):
        in_specs.append(pl.BlockSpec(a.shape, lambda i: (0, 0)))
        args.append(a)
    half = jax.ShapeDtypeStruct((E_pad, D_out // 2), jnp.float32)
    return pl.pallas_call(
        kern, grid=(NBLK,), in_specs=in_specs,
        out_specs=[pl.BlockSpec((B, D_out // 2), lambda i: (i, 0))] * 2,
        out_shape=[half, half])(*args)


def _node_stage(table, parts, u, lyrs, dv, da, B, final=None):
    N, Dtab = table.shape
    Dagg = parts[0].shape[1]
    W1 = lyrs[0]["W"]
    H1 = W1.shape[0]
    wts = [_pad2(W1[:, 0:dv].T, Dtab, H1),
           _pad2(W1[:, dv:dv + da].T, Dagg, H1),
           lyrs[0]["b"][None, :],
           W1[:, -1][None, :]]
    for lyr in lyrs[1:]:
        wts += [lyr["W"].T, lyr["b"][None, :]]
    D_out = lyrs[-1]["W"].shape[0]
    if final is not None:
        wts += [_pad2(final["W"].T, final["W"].shape[1], 8),
                _pad2(final["b"][None, :], 1, 8)]
        D_out = 8
    nw = len(wts)
    nparts = len(parts)
    NBLK = N // B

    def kern(*refs):
        t_ref = refs[0]
        agg = refs[1][...]
        for j in range(2, 1 + nparts):
            agg = agg + refs[j][...]
        u_ref = refs[1 + nparts]
        w = [r[...] for r in refs[2 + nparts:2 + nparts + nw]]
        out_ref = refs[2 + nparts + nw]
        h = (_dot(t_ref[...], w[0]) + _dot(agg, w[1]) + w[2]
             + _b(u_ref[0, 0]) * _b(w[3]))
        h = jnp.maximum(h, 0.0)
        nl = len(lyrs) - 1
        k = 4
        for _ in range(nl):
            h = jnp.maximum(_dot(h, w[k]) + w[k + 1], 0.0)
            k += 2
        if final is not None:
            h = _dot(h, w[k]) + w[k + 1]
        out_ref[...] = h

    in_specs = [pl.BlockSpec((B, Dtab), lambda i: (i, 0))]
    args = [table]
    for p in parts:
        in_specs.append(pl.BlockSpec((B, Dagg), lambda i: (i, 0)))
        args.append(p)
    for a in (u, *wts):
        in_specs.append(pl.BlockSpec(a.shape, lambda i: (0, 0)))
        args.append(a)
    return pl.pallas_call(
        kern, grid=(NBLK,), in_specs=in_specs,
        out_specs=pl.BlockSpec((B, D_out), lambda i: (i, 0)),
        out_shape=jax.ShapeDtypeStruct((N, D_out), jnp.float32))(*args)


@functools.cache
def _sc_mesh():
    return plsc.VectorSubcoreMesh(core_axis_name="c", subcore_axis_name="s")


def _sc_gather_pair(table, idx_s, idx_r, CH):
    E_pad = idx_s.shape[0]
    N, D = table.shape
    per_w = E_pad // 32
    nch = per_w // CH

    @functools.partial(
        pl.kernel,
        out_type=[jax.ShapeDtypeStruct((E_pad, D), jnp.float32),
                  jax.ShapeDtypeStruct((E_pad, D), jnp.float32)],
        mesh=_sc_mesh(),
        compiler_params=pltpu.CompilerParams(use_tc_tiling_on_sc=False),
        scratch_types=[pltpu.VMEM((2, CH), jnp.int32),
                       pltpu.VMEM((2, CH), jnp.int32),
                       pltpu.VMEM((2, CH, D), jnp.float32),
                       pltpu.VMEM((2, CH, D), jnp.float32)]
        + [pltpu.SemaphoreType.DMA] * 6)
    def k(tab, isrc, irsc, outs, outr, iv_s, iv_r, rv_s, rv_r,
          si, sg, sw, si1, sg1, sw1):
        wid = lax.axis_index("s") * 2 + lax.axis_index("c")
        base = wid * per_w
        sem_i, sem_g, sem_w = (si, si1), (sg, sg1), (sw, sw1)

        def start_idx(j, b):
            off = base + j * CH
            return (pltpu.async_copy(isrc.at[pl.ds(off, CH)], iv_s.at[b],
                                     sem_i[b]),
                    pltpu.async_copy(irsc.at[pl.ds(off, CH)], iv_r.at[b],
                                     sem_i[b]))

        idx_d = [None, None]
        wb_d = [None, None]
        idx_d[0] = start_idx(0, 0)
        for j in range(nch):
            b = j & 1
            nb = 1 - b
            if j + 1 < nch:
                idx_d[nb] = start_idx(j + 1, nb)
            for c in idx_d[b]:
                c.wait()
            if wb_d[b] is not None:
                for c in wb_d[b]:
                    c.wait()
            cs = pltpu.async_copy(tab.at[iv_s.at[b]], rv_s.at[b], sem_g[b])
            cr = pltpu.async_copy(tab.at[iv_r.at[b]], rv_r.at[b], sem_g[b])
            cs.wait()
            cr.wait()
            off = base + j * CH
            wb_d[b] = (pltpu.async_copy(rv_s.at[b], outs.at[pl.ds(off, CH)],
                                        sem_w[b]),
                       pltpu.async_copy(rv_r.at[b], outr.at[pl.ds(off, CH)],
                                        sem_w[b]))
        for b in (0, 1):
            if wb_d[b] is not None:
                for c in wb_d[b]:
                    c.wait()

    return k(table, idx_s, idx_r)


def _sc_scatter_add(rows_list, idx_list, N, D, CH):
    stripe = N // 16
    per_w = [r.shape[0] // 32 for r in rows_list]
    nch = [p // CH for p in per_w]
    zeros = jnp.zeros((stripe, D), jnp.float32)

    seq = [(a, i) for a in range(len(rows_list)) for i in range(nch[a])]

    @functools.partial(
        pl.kernel,
        out_type=jax.ShapeDtypeStruct((2 * N, D), jnp.float32),
        mesh=_sc_mesh(),
        compiler_params=pltpu.CompilerParams(use_tc_tiling_on_sc=False),
        scratch_types=[pltpu.VMEM((2, CH), jnp.int32),
                       pltpu.VMEM((2, CH, D), jnp.float32),
                       pltpu.VMEM_SHARED((N, D), jnp.float32)]
        + [pltpu.SemaphoreType.DMA] * 4)
    def k(*refs):
        na = len(rows_list)
        rows = refs[0:na]
        idxs = refs[na:2 * na]
        zref = refs[2 * na]
        out = refs[2 * na + 1]
        iv, rv, acc, sl0, sl1, ss0, ss1 = refs[2 * na + 2:2 * na + 9]
        sem_l, sem_s = (sl0, sl1), (ss0, ss1)
        c = lax.axis_index("c")
        s = lax.axis_index("s")
        pltpu.sync_copy(zref, acc.at[pl.ds(s * stripe, stripe)])
        plsc.subcore_barrier()
        wid = s * 2 + c

        def start_load(j, b):
            a, i = seq[j]
            off = wid * per_w[a] + i * CH
            return (pltpu.async_copy(idxs[a].at[pl.ds(off, CH)], iv.at[b],
                                     sem_l[b]),
                    pltpu.async_copy(rows[a].at[pl.ds(off, CH)], rv.at[b],
                                     sem_l[b]))

        ld_d = [None, None]
        sc_d = [None, None]
        ld_d[0] = start_load(0, 0)
        for j in range(len(seq)):
            b = j & 1
            nb = 1 - b
            if j + 1 < len(seq):
                if sc_d[nb] is not None:
                    sc_d[nb].wait()
                    sc_d[nb] = None
                ld_d[nb] = start_load(j + 1, nb)
            for cpy in ld_d[b]:
                cpy.wait()
            if sc_d[b] is not None:
                sc_d[b].wait()
            sc_d[b] = pltpu.async_copy(rv.at[b], acc.at[iv.at[b]], sem_s[b],
                                       add=True)
        for b in (0, 1):
            if sc_d[b] is not None:
                sc_d[b].wait()
        plsc.subcore_barrier()
        pltpu.sync_copy(acc.at[pl.ds(s * stripe, stripe)],
                        out.at[pl.ds(c * N + s * stripe, stripe)])

    out = k(*rows_list, *idx_list, zeros)
    return out[0:N] + out[N:]


_NP_COLS = (0, 3, 4)


def kernel(V, R_s, R_r, assignments_0, assignments_1, V_supers_0, V_supers_1,
           super_graphs_0, super_graphs_1, dt, params):
    V = V[0]
    R_s = R_s[0].astype(jnp.int32)
    R_r = R_r[0].astype(jnp.int32)
    a0 = assignments_0[0].astype(jnp.int32)
    a1 = assignments_1[0].astype(jnp.int32)
    Vs0 = V_supers_0[0]
    Vs1 = V_supers_1[0]
    sg0 = super_graphs_0[0].astype(jnp.int32)
    sg1 = super_graphs_1[0].astype(jnp.int32)
    u = dt.reshape(1, 1).astype(jnp.float32)
    npc = jnp.array(_NP_COLS)

    Vnp, Vpos = V[:, npc], V[:, 1:3]
    Vs0np, Vs0pos = Vs0[:, npc], Vs0[:, 1:3]
    Vs1np, Vs1pos = Vs1[:, npc], Vs1[:, 1:3]
    Vtab = _pad2(jnp.concatenate([Vnp, Vpos], 1), 10240, 16)
    Vs0tab = _pad2(jnp.concatenate([Vs0np, Vs0pos], 1), 1024, 16)
    Vs1tab = _pad2(jnp.concatenate([Vs1np, Vs1pos], 1), 128, 16)

    w1 = _prep_edge_w(params["edge_to_super"], 3, 3, 16, 16, 112)
    agg1 = _edge_onehot(_padi(a0[:, 1], 10240, 0), _padi(a0[:, 0], 10240, 0),
                        Vtab[0:1024], Vs0tab, u, w1, 3, 3, B=1024, D_out=112,
                        idx_o=_padi(a0[:, 0], 10240, 1023), NB_out=1024)
    VLtab = _pad2(jnp.concatenate([Vs0np, agg1[0:1000, 0:100], Vs0pos], 1),
                  1024, 112)

    w2 = _prep_edge_w(params["edge_to_upper"], 103, 3, 112, 16, 112)
    agg2 = _edge_onehot(_padi(a1[:, 1], 1024, 0), _padi(a1[:, 0], 1024, 0),
                        VLtab[0:128], Vs1tab, u, w2, 103, 3, B=512, D_out=112,
                        idx_o=_padi(a1[:, 0], 1024, 127), NB_out=128)
    Vtoptab = _pad2(jnp.concatenate([Vs1np, agg2[0:100, 0:100], Vs1pos], 1),
                    128, 112)

    w3 = _prep_edge_w(params["super_edge"], 103, 103, 112, 112, 160)
    agg3 = _edge_onehot(_padi(sg1[:, 0], 2048, 0), _padi(sg1[:, 1], 2048, 0),
                        Vtoptab, Vtoptab, u, w3, 103, 103, B=512, D_out=160,
                        idx_o=_padi(sg1[:, 1], 2048, 127), NB_out=128)
    vnew3 = _node_stage(Vtoptab, [agg3], u, params["super_node"], 103, 150,
                        B=128)
    Vuptab = _pad2(jnp.concatenate([Vs1np, vnew3[0:100, 0:100], Vs1pos], 1),
                   128, 112)

    w4a = _prep_edge_w(params["edge_from_upper"], 103, 103, 112, 112, 160)
    p4a = _edge_onehot(_padi(a1[:, 0], 1024, 0), _padi(a1[:, 1], 1024, 0),
                       Vuptab, VLtab[0:128], u, w4a, 103, 103, B=512,
                       D_out=160, idx_o=_padi(a1[:, 1], 1024, 1023),
                       NB_out=1024)
    p4b = _edge_onehot(_padi(sg0[:, 0], 16384, 0), _padi(sg0[:, 1], 16384, 0),
                       VLtab, VLtab, u, w3, 103, 103, B=4096, D_out=160,
                       idx_o=_padi(sg0[:, 1], 16384, 1023), NB_out=1024)
    vnew4 = _node_stage(VLtab, [p4a, p4b], u, params["super_node"], 103, 150,
                        B=512)
    Vup2tab = _pad2(jnp.concatenate([Vs0np, vnew4[0:1000, 0:100], Vs0pos], 1),
                    1024, 112)

    gs, gr = _sc_gather_pair(Vtab, _padi(R_s, 163840, 0),
                             _padi(R_r, 163840, 0), CH=1280)
    w5e = _prep_edge_w(params["edge"], 3, 3, 16, 16, 160)
    en_lo, en_hi = _edge_rows(gs, gr, u, w5e, 3, 3, B=8192, D_out=160)
    w5a = _prep_edge_w(params["edge_from_super"], 103, 3, 112, 16, 160)
    ui_lo, ui_hi = _edge_onehot(_padi(a0[:, 0], 10240, 0),
                                _padi(a0[:, 1], 10240, 0),
                                Vup2tab, Vtab[0:1024], u, w5a, 103, 3, B=5120,
                                D_out=160)
    idx1 = _padi(R_r, 163840, 10200)
    idx2 = _padi(a0[:, 1], 10240, 10200)
    agg_lo = _sc_scatter_add([en_lo, ui_lo], [idx1, idx2], N=10240, D=80,
                             CH=320)
    agg_hi = _sc_scatter_add([en_hi, ui_hi], [idx1, idx2], N=10240, D=80,
                             CH=320)
    agg5 = jnp.concatenate([agg_lo, agg_hi], 1)
    out5 = _node_stage(Vtab, [agg5], u,
                       params["node"], 3, 150, B=512, final=params["linear"])
    return out5[0:10000, 0:4][None]
